# baseline (speedup 1.0000x reference)
import jax
import jax.numpy as jnp
from jax import lax
from jax.experimental import pallas as pl
from jax.experimental.pallas import tpu as pltpu
from jax.experimental.pallas import tpu_sc as plsc

N, E, ND, ED, H, NCONV, B = 10000, 320000, 128, 16, 128, 3, 16
NSC = 2
NSUB = 16
NW = NSC * NSUB
EPW = E // NW
C = 16
NCHUNK = EPW // C
NP_ = 10240
RPT = NP_ // NSUB
ZR = 128

_INTERPRET = False


def _mm_kernel(x_ref, w_ref, b_ref, o_ref):
    o_ref[...] = (
        jnp.dot(x_ref[...], w_ref[...], preferred_element_type=jnp.float32)
        + b_ref[...]
    )


def _mm(x, w, b, bm):
    m, k = x.shape
    kout = w.shape[1]
    return pl.pallas_call(
        _mm_kernel,
        grid=(m // bm,),
        in_specs=[
            pl.BlockSpec((bm, k), lambda i: (i, 0)),
            pl.BlockSpec((k, kout), lambda i: (0, 0)),
            pl.BlockSpec((1, kout), lambda i: (0, 0)),
        ],
        out_specs=pl.BlockSpec((bm, kout), lambda i: (i, 0)),
        out_shape=jax.ShapeDtypeStruct((m, kout), jnp.float32),
        interpret=_INTERPRET,
    )(x, w, b.reshape(1, kout))


def _proj_kernel(h_ref, ws_ref, wd_ref, os_ref, od_ref):
    hh = h_ref[...]
    os_ref[...] = jnp.dot(hh, ws_ref[...], preferred_element_type=jnp.float32)
    od_ref[...] = jnp.dot(hh, wd_ref[...], preferred_element_type=jnp.float32)


def _proj(h, wsrc, wdst, bm):
    return pl.pallas_call(
        _proj_kernel,
        grid=(N // bm,),
        in_specs=[
            pl.BlockSpec((bm, H), lambda i: (i, 0)),
            pl.BlockSpec((H, 2 * H), lambda i: (0, 0)),
            pl.BlockSpec((H, 2 * H), lambda i: (0, 0)),
        ],
        out_specs=[
            pl.BlockSpec((bm, 2 * H), lambda i: (i, 0)),
            pl.BlockSpec((bm, 2 * H), lambda i: (i, 0)),
        ],
        out_shape=[
            jax.ShapeDtypeStruct((N, 2 * H), jnp.float32),
            jax.ShapeDtypeStruct((N, 2 * H), jnp.float32),
        ],
        interpret=_INTERPRET,
    )(h, wsrc, wdst)


def _make_edge_kernel():
    def body(src, dst, tsrc, tdst, ecg, out,
             idx_s0, idx_s1, idx_d0, idx_d1, idx_c0, idx_c1,
             rows_s0, rows_s1, rows_d0, rows_d1, ecg0, ecg1,
             msg0, msg1, acc, sem_i0, sem_i1, sem_g0, sem_g1):
        cid = lax.axis_index("c")
        sid = lax.axis_index("s")
        wid = sid * NSC + cid
        base0 = wid * EPW

        idx_s = [idx_s0, idx_s1]
        idx_d = [idx_d0, idx_d1]
        idx_c = [idx_c0, idx_c1]
        rows_s = [rows_s0, rows_s1]
        rows_d = [rows_d0, rows_d1]
        ecg_v = [ecg0, ecg1]
        msg_v = [msg0, msg1]
        sem_i = [sem_i0, sem_i1]
        sem_g = [sem_g0, sem_g1]

        def zb(i, _):
            r = i // 8
            msg0[r, pl.ds((i % 8) * 16, 16)] = jnp.zeros((16,), jnp.float32)
            return 0

        lax.fori_loop(0, C * 8, zb, 0)
        for k in range(RPT // C):
            pltpu.sync_copy(msg0, acc.at[pl.ds(sid * RPT + k * C, C)])
        plsc.subcore_barrier()

        def fire_idx(b, k):
            base = base0 + k * C
            pltpu.async_copy(src.at[pl.ds(base, C)], idx_s[b], sem_i[b])
            pltpu.async_copy(dst.at[pl.ds(base, C)], idx_d[b], sem_i[b])

        def wait_idx(b):
            pltpu.make_async_copy(src.at[pl.ds(0, C)], idx_s[b], sem_i[b]).wait()
            pltpu.make_async_copy(dst.at[pl.ds(0, C)], idx_d[b], sem_i[b]).wait()

        def fire_gather(b, k):
            base = base0 + k * C
            pltpu.async_copy(tsrc.at[idx_s[b]], rows_s[b], sem_g[b])
            pltpu.async_copy(tdst.at[idx_d[b]], rows_d[b], sem_g[b])
            pltpu.async_copy(ecg.at[pl.ds(base, C)], ecg_v[b], sem_g[b])

        def wait_gather(b):
            pltpu.make_async_copy(tsrc.at[idx_s[b]], rows_s[b], sem_g[b]).wait()
            pltpu.make_async_copy(tdst.at[idx_d[b]], rows_d[b], sem_g[b]).wait()
            pltpu.make_async_copy(ecg.at[pl.ds(0, C)], ecg_v[b], sem_g[b]).wait()

        def compute_scatter(b):
            rs, rd, ev, mv = rows_s[b], rows_d[b], ecg_v[b], msg_v[b]

            def edge_body(e, _):
                for j in range(8):
                    a = rs[e, pl.ds(j * 16, 16)]
                    bb = rd[e, pl.ds(j * 16, 16)]
                    ec = ev[e, pl.ds(j * 16, 16)]
                    cc = rs[e, pl.ds(128 + j * 16, 16)]
                    dd = rd[e, pl.ds(128 + j * 16, 16)]
                    eg = ev[e, pl.ds(128 + j * 16, 16)]
                    pre_c = a + bb + ec
                    pre_g = cc + dd + eg
                    den = (1.0 + jnp.exp(-pre_c)) * (1.0 + jnp.exp(-pre_g))
                    mv[e, pl.ds(j * 16, 16)] = pre_c / den
                return 0

            lax.fori_loop(0, C, edge_body, 0, unroll=4)
            pltpu.sync_copy(mv, acc.at[idx_c[b]], add=True)

        def phase(b, k, do_fi, do_fg):
            wait_gather(b)
            idx_c[b][pl.ds(0, 16)] = idx_d[b][pl.ds(0, 16)]
            if do_fi:
                fire_idx(b, k + 2)
            if do_fg:
                wait_idx(1 - b)
                fire_gather(1 - b, k + 1)
            compute_scatter(b)

        fire_idx(0, 0)
        wait_idx(0)
        fire_gather(0, 0)
        fire_idx(1, 1)

        def pair_body(i, _):
            k0 = 2 * i
            phase(0, k0, True, True)
            phase(1, k0 + 1, True, True)
            return 0

        lax.fori_loop(0, NCHUNK // 2 - 1, pair_body, 0)
        phase(0, NCHUNK - 3, True, True)
        phase(1, NCHUNK - 2, False, True)
        phase(0, NCHUNK - 1, False, False)
        plsc.subcore_barrier()

        for k in range(RPT // C):
            r0 = sid * RPT + k * C
            pltpu.sync_copy(acc.at[pl.ds(r0, C)], msg0)
            pltpu.sync_copy(msg0, out.at[cid, pl.ds(r0, C)])

    scratch = (
        [pltpu.VMEM((C,), jnp.int32) for _ in range(6)]
        + [pltpu.VMEM((C, 2 * H), jnp.float32) for _ in range(6)]
        + [pltpu.VMEM((C, H), jnp.float32) for _ in range(2)]
        + [pltpu.VMEM_SHARED((NP_, H), jnp.float32)]
        + [pltpu.SemaphoreType.DMA for _ in range(4)]
    )
    mesh = plsc.VectorSubcoreMesh(
        core_axis_name="c", subcore_axis_name="s",
        num_cores=NSC, num_subcores=NSUB,
    )
    return pl.kernel(
        body,
        out_type=jax.ShapeDtypeStruct((NSC, NP_, H), jnp.float32),
        mesh=mesh,
        scratch_types=scratch,
        interpret=_INTERPRET,
    )


CD = 40


def _make_deg_kernel():
    def body(dst, outd, idx_d, ones_v, bnc, dacc, sem0):
        cid = lax.axis_index("c")
        sid = lax.axis_index("s")
        wid = sid * NSC + cid

        def zb(i, _):
            r = i // 8
            col = (i % 8) * 16
            bnc[r, pl.ds(col, 16)] = jnp.zeros((16,), jnp.float32)
            return 0

        lax.fori_loop(0, CD * 8, zb, 0)

        def ob(i, _):
            r = i // 8
            col = (i % 8) * 16
            ones_v[r, pl.ds(col, 16)] = jnp.full((16,), 1.0 / 128.0, jnp.float32)
            return 0

        lax.fori_loop(0, CD * 8, ob, 0)
        for k in range(RPT // CD):
            pltpu.sync_copy(bnc, dacc.at[pl.ds(sid * RPT + k * CD, CD)])
        plsc.subcore_barrier()

        def chunk_body(k, _):
            base = wid * EPW + k * CD
            pltpu.sync_copy(dst.at[pl.ds(base, CD)], idx_d)
            pltpu.sync_copy(ones_v, dacc.at[idx_d], add=True)
            return 0

        lax.fori_loop(0, EPW // CD, chunk_body, 0)
        plsc.subcore_barrier()
        for k in range(RPT // CD):
            r0 = sid * RPT + k * CD
            pltpu.sync_copy(dacc.at[pl.ds(r0, CD)], bnc)
            pltpu.sync_copy(bnc, outd.at[cid, pl.ds(r0, CD)])

    scratch = [
        pltpu.VMEM((CD,), jnp.int32),
        pltpu.VMEM((CD, H), jnp.float32),
        pltpu.VMEM((CD, H), jnp.float32),
        pltpu.VMEM_SHARED((NP_, H), jnp.float32),
        pltpu.SemaphoreType.DMA,
    ]
    mesh = plsc.VectorSubcoreMesh(
        core_axis_name="c", subcore_axis_name="s",
        num_cores=NSC, num_subcores=NSUB,
    )
    return pl.kernel(
        body,
        out_type=jax.ShapeDtypeStruct((NSC, NP_, H), jnp.float32),
        mesh=mesh,
        scratch_types=scratch,
        interpret=_INTERPRET,
    )


def _combine_kernel(hs_ref, dp_ref, x_ref, st_ref):
    s = hs_ref[0] + hs_ref[1]
    deg = jnp.clip(
        jnp.sum(dp_ref[0] + dp_ref[1], axis=-1, keepdims=True), 1.0, None
    )
    x = s / deg
    x_ref[...] = x

    @pl.when(pl.program_id(0) == 0)
    def _():
        st_ref[...] = jnp.zeros_like(st_ref)

    st_ref[0:1, :] += jnp.sum(x, axis=0, keepdims=True)
    st_ref[1:2, :] += jnp.sum(x * x, axis=0, keepdims=True)


def _combine(hsum, dpart, bm):
    return pl.pallas_call(
        _combine_kernel,
        grid=(N // bm,),
        in_specs=[
            pl.BlockSpec((NSC, bm, H), lambda i: (0, i, 0)),
            pl.BlockSpec((NSC, bm, H), lambda i: (0, i, 0)),
        ],
        out_specs=[
            pl.BlockSpec((bm, H), lambda i: (i, 0)),
            pl.BlockSpec((8, H), lambda i: (0, 0)),
        ],
        out_shape=[
            jax.ShapeDtypeStruct((N, H), jnp.float32),
            jax.ShapeDtypeStruct((8, H), jnp.float32),
        ],
        interpret=_INTERPRET,
    )(hsum, dpart)


def _bnres_kernel(x_ref, st_ref, g_ref, b_ref, hp_ref, o_ref):
    mean = st_ref[0:1, :] / N
    var = st_ref[1:2, :] / N - mean * mean
    xn = (x_ref[...] - mean) * lax.rsqrt(var + 1e-5) * g_ref[...] + b_ref[...]
    t = hp_ref[...] + xn
    o_ref[...] = t * (1.0 / (1.0 + jnp.exp(-t)))


def _bnres(x, stats, g, b, hprev, bm):
    return pl.pallas_call(
        _bnres_kernel,
        grid=(N // bm,),
        in_specs=[
            pl.BlockSpec((bm, H), lambda i: (i, 0)),
            pl.BlockSpec((8, H), lambda i: (0, 0)),
            pl.BlockSpec((1, H), lambda i: (0, 0)),
            pl.BlockSpec((1, H), lambda i: (0, 0)),
            pl.BlockSpec((bm, H), lambda i: (i, 0)),
        ],
        out_specs=pl.BlockSpec((bm, H), lambda i: (i, 0)),
        out_shape=jax.ShapeDtypeStruct((N, H), jnp.float32),
        interpret=_INTERPRET,
    )(x, stats, g.reshape(1, H), b.reshape(1, H), hprev)


def _tail_kernel(h0, h1, h2, h3, w0, w1, w2, w3, jkb, oh,
                 fw0, fb0, bg, bb, fw1, fb1, o_ref, gsum, gmax, cnt):
    i = pl.program_id(0)
    hjk = (
        jnp.dot(h0[...], w0[...], preferred_element_type=jnp.float32)
        + jnp.dot(h1[...], w1[...], preferred_element_type=jnp.float32)
        + jnp.dot(h2[...], w2[...], preferred_element_type=jnp.float32)
        + jnp.dot(h3[...], w3[...], preferred_element_type=jnp.float32)
        + jkb[...]
    )

    @pl.when(i == 0)
    def _():
        gsum[...] = jnp.zeros_like(gsum)
        gmax[...] = jnp.full_like(gmax, -jnp.inf)
        cnt[...] = jnp.zeros_like(cnt)

    ohb = oh[...]
    gsum[...] += lax.dot_general(
        ohb, hjk, (((0,), (0,)), ((), ())),
        preferred_element_type=jnp.float32,
    )
    cnt[...] += lax.dot_general(
        ohb, jnp.ones_like(hjk), (((0,), (0,)), ((), ())),
        preferred_element_type=jnp.float32,
    )
    for b in range(B):
        m = ohb[:, b:b + 1] > 0.5
        mx = jnp.max(jnp.where(m, hjk, -jnp.inf), axis=0, keepdims=True)
        gmax[b:b + 1, :] = jnp.maximum(gmax[b:b + 1, :], mx)

    @pl.when(i == pl.num_programs(0) - 1)
    def _():
        c = jnp.clip(cnt[...], 1.0, None)
        gmean = gsum[...] / c
        gm = gmax[...]
        gm = jnp.where(gm > -1e30, gm, 0.0)
        g = jnp.concatenate([gmean, gm], axis=1)
        x = jnp.dot(g, fw0[...], preferred_element_type=jnp.float32) + fb0[...]
        mean = jnp.mean(x, axis=0, keepdims=True)
        var = jnp.mean(x * x, axis=0, keepdims=True) - mean * mean
        xn = (x - mean) * lax.rsqrt(var + 1e-5) * bg[...] + bb[...]
        xs = xn * (1.0 / (1.0 + jnp.exp(-xn)))
        o_ref[...] = (
            jnp.dot(xs, fw1[...], preferred_element_type=jnp.float32)
            + fb1[...]
        )


def _tail(states, jk_W, jk_b, onehot, fc0_W, fc0_b, bg, bb, fw1p, fb1p, bm):
    w_specs = [pl.BlockSpec((H, H), lambda i: (0, 0)) for _ in range(4)]
    return pl.pallas_call(
        _tail_kernel,
        grid=(N // bm,),
        in_specs=(
            [pl.BlockSpec((bm, H), lambda i: (i, 0)) for _ in range(4)]
            + w_specs
            + [
                pl.BlockSpec((1, H), lambda i: (0, 0)),
                pl.BlockSpec((bm, B), lambda i: (i, 0)),
                pl.BlockSpec((2 * H, H), lambda i: (0, 0)),
                pl.BlockSpec((1, H), lambda i: (0, 0)),
                pl.BlockSpec((1, H), lambda i: (0, 0)),
                pl.BlockSpec((1, H), lambda i: (0, 0)),
                pl.BlockSpec((H, H), lambda i: (0, 0)),
                pl.BlockSpec((1, H), lambda i: (0, 0)),
            ]
        ),
        out_specs=pl.BlockSpec((B, H), lambda i: (0, 0)),
        out_shape=jax.ShapeDtypeStruct((B, H), jnp.float32),
        scratch_shapes=[
            pltpu.VMEM((B, H), jnp.float32),
            pltpu.VMEM((B, H), jnp.float32),
            pltpu.VMEM((B, H), jnp.float32),
        ],
        interpret=_INTERPRET,
    )(
        states[0], states[1], states[2], states[3],
        jk_W[0:H], jk_W[H:2 * H], jk_W[2 * H:3 * H], jk_W[3 * H:4 * H],
        jk_b.reshape(1, H), onehot,
        fc0_W, fc0_b.reshape(1, H), bg.reshape(1, H), bb.reshape(1, H),
        fw1p, fb1p.reshape(1, H),
    )


def kernel(node_feats, edge_index, edge_feats, batch, node_embed_W,
           node_embed_b, conv0_Wc, conv0_bc, conv0_Wg, conv0_bg, bn0_g,
           bn0_b, conv1_Wc, conv1_bc, conv1_Wg, conv1_bg, bn1_g, bn1_b,
           conv2_Wc, conv2_bc, conv2_Wg, conv2_bg, bn2_g, bn2_b, jk_W,
           jk_b, fc0_W, fc0_b, fc_bn_g, fc_bn_b, fc1_W, fc1_b):
    convs = [
        (conv0_Wc, conv0_bc, conv0_Wg, conv0_bg, bn0_g, bn0_b),
        (conv1_Wc, conv1_bc, conv1_Wg, conv1_bg, bn1_g, bn1_b),
        (conv2_Wc, conv2_bc, conv2_Wg, conv2_bg, bn2_g, bn2_b),
    ]
    bm = 1000
    src_idx = edge_index[0]
    dst_idx = edge_index[1]

    h = _mm(node_feats, node_embed_W, node_embed_b, bm)
    states = [h]
    edge_k = _make_edge_kernel()
    dpart = _make_deg_kernel()(dst_idx)
    for i, (Wc, bc, Wg, bg, bng, bnb) in enumerate(convs):
        wsrc = jnp.concatenate([Wc[0:H], Wg[0:H]], axis=1)
        wdst = jnp.concatenate([Wc[H:2 * H], Wg[H:2 * H]], axis=1)
        we = jnp.concatenate([Wc[2 * H:], Wg[2 * H:]], axis=1)
        bcat = jnp.concatenate([bc, bg])
        tsrc, tdst = _proj(h, wsrc, wdst, bm)
        ecg = _mm(edge_feats, we, bcat, bm)
        hsum = edge_k(src_idx, dst_idx, tsrc, tdst, ecg)
        x, stats = _combine(hsum, dpart, bm)
        h = _bnres(x, stats, bng, bnb, h, bm)
        states.append(h)

    onehot = (batch[:, None] == jnp.arange(B, dtype=jnp.int32)[None, :])
    onehot = onehot.astype(jnp.float32)
    fw1p = jnp.pad(fc1_W, ((0, 0), (0, H - 1)))
    fb1p = jnp.pad(fc1_b, (0, H - 1))
    out = _tail(states, jk_W, jk_b, onehot, fc0_W, fc0_b, fc_bn_g,
                fc_bn_b, fw1p, fb1p, bm)
    return out[:, 0:1]


# speedup vs baseline: 2.2145x; 2.2145x over previous
import jax
import jax.numpy as jnp
from jax import lax
from jax.experimental import pallas as pl
from jax.experimental.pallas import tpu as pltpu
from jax.experimental.pallas import tpu_sc as plsc

N, E, ND, ED, H, NCONV, B = 10000, 320000, 128, 16, 128, 3, 16
NSC = 2
NSUB = 16
NW = NSC * NSUB
EPW = E // NW
C = 40
NCHUNK = EPW // C
NP_ = 10240
RPT = NP_ // NSUB
ZR = 128

_INTERPRET = False


def _mm_kernel(x_ref, w_ref, b_ref, o_ref):
    o_ref[...] = (
        jnp.dot(x_ref[...], w_ref[...], preferred_element_type=jnp.float32)
        + b_ref[...]
    )


def _mm(x, w, b, bm):
    m, k = x.shape
    kout = w.shape[1]
    return pl.pallas_call(
        _mm_kernel,
        grid=(m // bm,),
        in_specs=[
            pl.BlockSpec((bm, k), lambda i: (i, 0)),
            pl.BlockSpec((k, kout), lambda i: (0, 0)),
            pl.BlockSpec((1, kout), lambda i: (0, 0)),
        ],
        out_specs=pl.BlockSpec((bm, kout), lambda i: (i, 0)),
        out_shape=jax.ShapeDtypeStruct((m, kout), jnp.float32),
        interpret=_INTERPRET,
    )(x, w, b.reshape(1, kout))


def _proj_kernel(h_ref, ws_ref, wd_ref, os_ref, od_ref):
    hh = h_ref[...]
    os_ref[...] = jnp.dot(hh, ws_ref[...], preferred_element_type=jnp.float32)
    od_ref[...] = jnp.dot(hh, wd_ref[...], preferred_element_type=jnp.float32)


def _proj(h, wsrc, wdst, bm):
    return pl.pallas_call(
        _proj_kernel,
        grid=(N // bm,),
        in_specs=[
            pl.BlockSpec((bm, H), lambda i: (i, 0)),
            pl.BlockSpec((H, 2 * H), lambda i: (0, 0)),
            pl.BlockSpec((H, 2 * H), lambda i: (0, 0)),
        ],
        out_specs=[
            pl.BlockSpec((bm, 2 * H), lambda i: (i, 0)),
            pl.BlockSpec((bm, 2 * H), lambda i: (i, 0)),
        ],
        out_shape=[
            jax.ShapeDtypeStruct((N, 2 * H), jnp.float32),
            jax.ShapeDtypeStruct((N, 2 * H), jnp.float32),
        ],
        interpret=_INTERPRET,
    )(h, wsrc, wdst)


def _make_gather_kernel():
    def body(src, dst, tsrc, tdst, out,
             idx_s0, idx_s1, idx_d0, idx_d1,
             rows_s0, rows_s1, rows_d0, rows_d1, sbuf0, sbuf1,
             sem_i0, sem_i1, sem_g0, sem_g1, sem_w0, sem_w1):
        cid = lax.axis_index("c")
        sid = lax.axis_index("s")
        wid = sid * NSC + cid
        base0 = wid * EPW

        idx_s = [idx_s0, idx_s1]
        idx_d = [idx_d0, idx_d1]
        rows_s = [rows_s0, rows_s1]
        rows_d = [rows_d0, rows_d1]
        sbuf = [sbuf0, sbuf1]
        sem_i = [sem_i0, sem_i1]
        sem_g = [sem_g0, sem_g1]
        sem_w = [sem_w0, sem_w1]

        def fire_idx(b, k):
            base = base0 + k * C
            pltpu.async_copy(src.at[pl.ds(base, C)], idx_s[b], sem_i[b])
            pltpu.async_copy(dst.at[pl.ds(base, C)], idx_d[b], sem_i[b])

        def wait_idx(b):
            pltpu.make_async_copy(src.at[pl.ds(0, C)], idx_s[b], sem_i[b]).wait()
            pltpu.make_async_copy(dst.at[pl.ds(0, C)], idx_d[b], sem_i[b]).wait()

        def fire_gather(b, k):
            pltpu.async_copy(tsrc.at[idx_s[b]], rows_s[b], sem_g[b])
            pltpu.async_copy(tdst.at[idx_d[b]], rows_d[b], sem_g[b])

        def wait_gather(b):
            pltpu.make_async_copy(tsrc.at[idx_s[b]], rows_s[b], sem_g[b]).wait()
            pltpu.make_async_copy(tdst.at[idx_d[b]], rows_d[b], sem_g[b]).wait()

        def fire_write(b, k):
            base = base0 + k * C
            pltpu.async_copy(sbuf[b], out.at[pl.ds(base, C)], sem_w[b])

        def wait_write(b):
            pltpu.make_async_copy(sbuf[b], out.at[pl.ds(0, C)], sem_w[b]).wait()

        def phase(b, k, ww, fi, fg):
            wait_gather(b)
            if ww:
                wait_write(b)
            if fi:
                fire_idx(b, k + 2)
            if fg:
                wait_idx(1 - b)
                fire_gather(1 - b, k + 1)
            rs, rd, sb = rows_s[b], rows_d[b], sbuf[b]

            def edge_body(e, _):
                for j in range(16):
                    sb[e, pl.ds(j * 16, 16)] = (
                        rs[e, pl.ds(j * 16, 16)] + rd[e, pl.ds(j * 16, 16)]
                    )
                return 0

            lax.fori_loop(0, C, edge_body, 0, unroll=2)
            fire_write(b, k)

        fire_idx(0, 0)
        wait_idx(0)
        fire_gather(0, 0)
        fire_idx(1, 1)
        phase(0, 0, False, True, True)
        phase(1, 1, False, True, True)

        def pair_body(i, _):
            k0 = 2 * i
            phase(0, k0, True, True, True)
            phase(1, k0 + 1, True, True, True)
            return 0

        lax.fori_loop(1, NCHUNK // 2 - 1, pair_body, 0)
        phase(0, NCHUNK - 2, True, False, True)
        phase(1, NCHUNK - 1, True, False, False)
        wait_write(0)
        wait_write(1)

    scratch = (
        [pltpu.VMEM((C,), jnp.int32) for _ in range(4)]
        + [pltpu.VMEM((C, 2 * H), jnp.float32) for _ in range(6)]
        + [pltpu.SemaphoreType.DMA for _ in range(6)]
    )
    mesh = plsc.VectorSubcoreMesh(
        core_axis_name="c", subcore_axis_name="s",
        num_cores=NSC, num_subcores=NSUB,
    )
    return pl.kernel(
        body,
        out_type=jax.ShapeDtypeStruct((E, 2 * H), jnp.float32),
        mesh=mesh,
        scratch_types=scratch,
        interpret=_INTERPRET,
    )


def _make_scatter_kernel():
    def body(dst, msg, out,
             idx_d0, idx_d1, mb0, mb1, acc, sem0, sem1):
        cid = lax.axis_index("c")
        sid = lax.axis_index("s")
        wid = sid * NSC + cid
        base0 = wid * EPW

        idx_d = [idx_d0, idx_d1]
        mb = [mb0, mb1]
        sem = [sem0, sem1]

        def zb(i, _):
            r = i // 8
            mb0[r, pl.ds((i % 8) * 16, 16)] = jnp.zeros((16,), jnp.float32)
            return 0

        lax.fori_loop(0, C * 8, zb, 0)
        for k in range(RPT // C):
            pltpu.sync_copy(mb0, acc.at[pl.ds(sid * RPT + k * C, C)])
        plsc.subcore_barrier()

        def fire_in(b, k):
            base = base0 + k * C
            pltpu.async_copy(dst.at[pl.ds(base, C)], idx_d[b], sem[b])
            pltpu.async_copy(msg.at[pl.ds(base, C)], mb[b], sem[b])

        def wait_in(b):
            pltpu.make_async_copy(dst.at[pl.ds(0, C)], idx_d[b], sem[b]).wait()
            pltpu.make_async_copy(msg.at[pl.ds(0, C)], mb[b], sem[b]).wait()

        def phase(b, k, ff):
            wait_in(b)
            if ff:
                fire_in(1 - b, k + 1)
            pltpu.sync_copy(mb[b], acc.at[idx_d[b]], add=True)

        fire_in(0, 0)

        def pair_body(i, _):
            k0 = 2 * i
            phase(0, k0, True)
            phase(1, k0 + 1, True)
            return 0

        lax.fori_loop(0, NCHUNK // 2 - 1, pair_body, 0)
        phase(0, NCHUNK - 2, True)
        phase(1, NCHUNK - 1, False)
        plsc.subcore_barrier()

        for k in range(RPT // C):
            r0 = sid * RPT + k * C
            pltpu.sync_copy(acc.at[pl.ds(r0, C)], mb0)
            pltpu.sync_copy(mb0, out.at[cid, pl.ds(r0, C)])

    scratch = (
        [pltpu.VMEM((C,), jnp.int32) for _ in range(2)]
        + [pltpu.VMEM((C, H), jnp.float32) for _ in range(2)]
        + [pltpu.VMEM_SHARED((NP_, H), jnp.float32)]
        + [pltpu.SemaphoreType.DMA for _ in range(2)]
    )
    mesh = plsc.VectorSubcoreMesh(
        core_axis_name="c", subcore_axis_name="s",
        num_cores=NSC, num_subcores=NSUB,
    )
    return pl.kernel(
        body,
        out_type=jax.ShapeDtypeStruct((NSC, NP_, H), jnp.float32),
        mesh=mesh,
        scratch_types=scratch,
        interpret=_INTERPRET,
    )


def _edge_nl_kernel(s_ref, ef_ref, we_ref, b_ref, o_ref):
    pre = (
        s_ref[...]
        + jnp.dot(ef_ref[...], we_ref[...], preferred_element_type=jnp.float32)
        + b_ref[...]
    )
    pc = pre[:, :H]
    pg = pre[:, H:]
    den = (1.0 + jnp.exp(-pc)) * (1.0 + jnp.exp(-pg))
    o_ref[...] = pc / den


def _edge_nl(s, ef, we, bcat, bm):
    return pl.pallas_call(
        _edge_nl_kernel,
        grid=(E // bm,),
        in_specs=[
            pl.BlockSpec((bm, 2 * H), lambda i: (i, 0)),
            pl.BlockSpec((bm, ED), lambda i: (i, 0)),
            pl.BlockSpec((ED, 2 * H), lambda i: (0, 0)),
            pl.BlockSpec((1, 2 * H), lambda i: (0, 0)),
        ],
        out_specs=pl.BlockSpec((bm, H), lambda i: (i, 0)),
        out_shape=jax.ShapeDtypeStruct((E, H), jnp.float32),
        interpret=_INTERPRET,
    )(s, ef, we, bcat.reshape(1, 2 * H))


CD = 40


def _make_deg_kernel():
    def body(dst, outd, idx_d, ones_v, bnc, dacc, sem0):
        cid = lax.axis_index("c")
        sid = lax.axis_index("s")
        wid = sid * NSC + cid

        def zb(i, _):
            r = i // 8
            col = (i % 8) * 16
            bnc[r, pl.ds(col, 16)] = jnp.zeros((16,), jnp.float32)
            return 0

        lax.fori_loop(0, CD * 8, zb, 0)

        def ob(i, _):
            r = i // 8
            col = (i % 8) * 16
            ones_v[r, pl.ds(col, 16)] = jnp.full((16,), 1.0 / 128.0, jnp.float32)
            return 0

        lax.fori_loop(0, CD * 8, ob, 0)
        for k in range(RPT // CD):
            pltpu.sync_copy(bnc, dacc.at[pl.ds(sid * RPT + k * CD, CD)])
        plsc.subcore_barrier()

        def chunk_body(k, _):
            base = wid * EPW + k * CD
            pltpu.sync_copy(dst.at[pl.ds(base, CD)], idx_d)
            pltpu.sync_copy(ones_v, dacc.at[idx_d], add=True)
            return 0

        lax.fori_loop(0, EPW // CD, chunk_body, 0)
        plsc.subcore_barrier()
        for k in range(RPT // CD):
            r0 = sid * RPT + k * CD
            pltpu.sync_copy(dacc.at[pl.ds(r0, CD)], bnc)
            pltpu.sync_copy(bnc, outd.at[cid, pl.ds(r0, CD)])

    scratch = [
        pltpu.VMEM((CD,), jnp.int32),
        pltpu.VMEM((CD, H), jnp.float32),
        pltpu.VMEM((CD, H), jnp.float32),
        pltpu.VMEM_SHARED((NP_, H), jnp.float32),
        pltpu.SemaphoreType.DMA,
    ]
    mesh = plsc.VectorSubcoreMesh(
        core_axis_name="c", subcore_axis_name="s",
        num_cores=NSC, num_subcores=NSUB,
    )
    return pl.kernel(
        body,
        out_type=jax.ShapeDtypeStruct((NSC, NP_, H), jnp.float32),
        mesh=mesh,
        scratch_types=scratch,
        interpret=_INTERPRET,
    )


def _combine_kernel(hs_ref, dp_ref, x_ref, st_ref):
    s = hs_ref[0] + hs_ref[1]
    deg = jnp.clip(
        jnp.sum(dp_ref[0] + dp_ref[1], axis=-1, keepdims=True), 1.0, None
    )
    x = s / deg
    x_ref[...] = x

    @pl.when(pl.program_id(0) == 0)
    def _():
        st_ref[...] = jnp.zeros_like(st_ref)

    st_ref[0:1, :] += jnp.sum(x, axis=0, keepdims=True)
    st_ref[1:2, :] += jnp.sum(x * x, axis=0, keepdims=True)


def _combine(hsum, dpart, bm):
    return pl.pallas_call(
        _combine_kernel,
        grid=(N // bm,),
        in_specs=[
            pl.BlockSpec((NSC, bm, H), lambda i: (0, i, 0)),
            pl.BlockSpec((NSC, bm, H), lambda i: (0, i, 0)),
        ],
        out_specs=[
            pl.BlockSpec((bm, H), lambda i: (i, 0)),
            pl.BlockSpec((8, H), lambda i: (0, 0)),
        ],
        out_shape=[
            jax.ShapeDtypeStruct((N, H), jnp.float32),
            jax.ShapeDtypeStruct((8, H), jnp.float32),
        ],
        interpret=_INTERPRET,
    )(hsum, dpart)


def _bnres_kernel(x_ref, st_ref, g_ref, b_ref, hp_ref, o_ref):
    mean = st_ref[0:1, :] / N
    var = st_ref[1:2, :] / N - mean * mean
    xn = (x_ref[...] - mean) * lax.rsqrt(var + 1e-5) * g_ref[...] + b_ref[...]
    t = hp_ref[...] + xn
    o_ref[...] = t * (1.0 / (1.0 + jnp.exp(-t)))


def _bnres(x, stats, g, b, hprev, bm):
    return pl.pallas_call(
        _bnres_kernel,
        grid=(N // bm,),
        in_specs=[
            pl.BlockSpec((bm, H), lambda i: (i, 0)),
            pl.BlockSpec((8, H), lambda i: (0, 0)),
            pl.BlockSpec((1, H), lambda i: (0, 0)),
            pl.BlockSpec((1, H), lambda i: (0, 0)),
            pl.BlockSpec((bm, H), lambda i: (i, 0)),
        ],
        out_specs=pl.BlockSpec((bm, H), lambda i: (i, 0)),
        out_shape=jax.ShapeDtypeStruct((N, H), jnp.float32),
        interpret=_INTERPRET,
    )(x, stats, g.reshape(1, H), b.reshape(1, H), hprev)


def _tail_kernel(h0, h1, h2, h3, w0, w1, w2, w3, jkb, oh,
                 fw0, fb0, bg, bb, fw1, fb1, o_ref, gsum, gmax, cnt):
    i = pl.program_id(0)
    hjk = (
        jnp.dot(h0[...], w0[...], preferred_element_type=jnp.float32)
        + jnp.dot(h1[...], w1[...], preferred_element_type=jnp.float32)
        + jnp.dot(h2[...], w2[...], preferred_element_type=jnp.float32)
        + jnp.dot(h3[...], w3[...], preferred_element_type=jnp.float32)
        + jkb[...]
    )

    @pl.when(i == 0)
    def _():
        gsum[...] = jnp.zeros_like(gsum)
        gmax[...] = jnp.full_like(gmax, -jnp.inf)
        cnt[...] = jnp.zeros_like(cnt)

    ohb = oh[...]
    gsum[...] += lax.dot_general(
        ohb, hjk, (((0,), (0,)), ((), ())),
        preferred_element_type=jnp.float32,
    )
    cnt[...] += lax.dot_general(
        ohb, jnp.ones_like(hjk), (((0,), (0,)), ((), ())),
        preferred_element_type=jnp.float32,
    )
    for b in range(B):
        m = ohb[:, b:b + 1] > 0.5
        mx = jnp.max(jnp.where(m, hjk, -jnp.inf), axis=0, keepdims=True)
        gmax[b:b + 1, :] = jnp.maximum(gmax[b:b + 1, :], mx)

    @pl.when(i == pl.num_programs(0) - 1)
    def _():
        c = jnp.clip(cnt[...], 1.0, None)
        gmean = gsum[...] / c
        gm = gmax[...]
        gm = jnp.where(gm > -1e30, gm, 0.0)
        g = jnp.concatenate([gmean, gm], axis=1)
        x = jnp.dot(g, fw0[...], preferred_element_type=jnp.float32) + fb0[...]
        mean = jnp.mean(x, axis=0, keepdims=True)
        var = jnp.mean(x * x, axis=0, keepdims=True) - mean * mean
        xn = (x - mean) * lax.rsqrt(var + 1e-5) * bg[...] + bb[...]
        xs = xn * (1.0 / (1.0 + jnp.exp(-xn)))
        o_ref[...] = (
            jnp.dot(xs, fw1[...], preferred_element_type=jnp.float32)
            + fb1[...]
        )


def _tail(states, jk_W, jk_b, onehot, fc0_W, fc0_b, bg, bb, fw1p, fb1p, bm):
    w_specs = [pl.BlockSpec((H, H), lambda i: (0, 0)) for _ in range(4)]
    return pl.pallas_call(
        _tail_kernel,
        grid=(N // bm,),
        in_specs=(
            [pl.BlockSpec((bm, H), lambda i: (i, 0)) for _ in range(4)]
            + w_specs
            + [
                pl.BlockSpec((1, H), lambda i: (0, 0)),
                pl.BlockSpec((bm, B), lambda i: (i, 0)),
                pl.BlockSpec((2 * H, H), lambda i: (0, 0)),
                pl.BlockSpec((1, H), lambda i: (0, 0)),
                pl.BlockSpec((1, H), lambda i: (0, 0)),
                pl.BlockSpec((1, H), lambda i: (0, 0)),
                pl.BlockSpec((H, H), lambda i: (0, 0)),
                pl.BlockSpec((1, H), lambda i: (0, 0)),
            ]
        ),
        out_specs=pl.BlockSpec((B, H), lambda i: (0, 0)),
        out_shape=jax.ShapeDtypeStruct((B, H), jnp.float32),
        scratch_shapes=[
            pltpu.VMEM((B, H), jnp.float32),
            pltpu.VMEM((B, H), jnp.float32),
            pltpu.VMEM((B, H), jnp.float32),
        ],
        interpret=_INTERPRET,
    )(
        states[0], states[1], states[2], states[3],
        jk_W[0:H], jk_W[H:2 * H], jk_W[2 * H:3 * H], jk_W[3 * H:4 * H],
        jk_b.reshape(1, H), onehot,
        fc0_W, fc0_b.reshape(1, H), bg.reshape(1, H), bb.reshape(1, H),
        fw1p, fb1p.reshape(1, H),
    )


def kernel(node_feats, edge_index, edge_feats, batch, node_embed_W,
           node_embed_b, conv0_Wc, conv0_bc, conv0_Wg, conv0_bg, bn0_g,
           bn0_b, conv1_Wc, conv1_bc, conv1_Wg, conv1_bg, bn1_g, bn1_b,
           conv2_Wc, conv2_bc, conv2_Wg, conv2_bg, bn2_g, bn2_b, jk_W,
           jk_b, fc0_W, fc0_b, fc_bn_g, fc_bn_b, fc1_W, fc1_b):
    convs = [
        (conv0_Wc, conv0_bc, conv0_Wg, conv0_bg, bn0_g, bn0_b),
        (conv1_Wc, conv1_bc, conv1_Wg, conv1_bg, bn1_g, bn1_b),
        (conv2_Wc, conv2_bc, conv2_Wg, conv2_bg, bn2_g, bn2_b),
    ]
    bm = 1000
    src_idx = edge_index[0]
    dst_idx = edge_index[1]

    h = _mm(node_feats, node_embed_W, node_embed_b, bm)
    states = [h]
    gat_k = _make_gather_kernel()
    sct_k = _make_scatter_kernel()
    dpart = _make_deg_kernel()(dst_idx)
    for i, (Wc, bc, Wg, bg, bng, bnb) in enumerate(convs):
        wsrc = jnp.concatenate([Wc[0:H], Wg[0:H]], axis=1)
        wdst = jnp.concatenate([Wc[H:2 * H], Wg[H:2 * H]], axis=1)
        we = jnp.concatenate([Wc[2 * H:], Wg[2 * H:]], axis=1)
        bcat = jnp.concatenate([bc, bg])
        tsrc, tdst = _proj(h, wsrc, wdst, bm)
        s = gat_k(src_idx, dst_idx, tsrc, tdst)
        msg = _edge_nl(s, edge_feats, we, bcat, bm)
        hsum = sct_k(dst_idx, msg)
        x, stats = _combine(hsum, dpart, bm)
        h = _bnres(x, stats, bng, bnb, h, bm)
        states.append(h)

    onehot = (batch[:, None] == jnp.arange(B, dtype=jnp.int32)[None, :])
    onehot = onehot.astype(jnp.float32)
    fw1p = jnp.pad(fc1_W, ((0, 0), (0, H - 1)))
    fb1p = jnp.pad(fc1_b, (0, H - 1))
    out = _tail(states, jk_W, jk_b, onehot, fc0_W, fc0_b, fc_bn_g,
                fc_bn_b, fw1p, fb1p, bm)
    return out[:, 0:1]


# speedup vs baseline: 2.2163x; 1.0008x over previous
import jax
import jax.numpy as jnp
from jax import lax
from jax.experimental import pallas as pl
from jax.experimental.pallas import tpu as pltpu
from jax.experimental.pallas import tpu_sc as plsc

N, E, ND, ED, H, NCONV, B = 10000, 320000, 128, 16, 128, 3, 16
NSC = 2
NSUB = 16
NW = NSC * NSUB
EPW = E // NW
C = 40
NCHUNK = EPW // C
NP_ = 10240
RPT = NP_ // NSUB
BM = 1000

_INTERPRET = False


def _embed_proj_kernel(x_ref, w_ref, b_ref, ws_ref, wd_ref,
                       h_ref, os_ref, od_ref):
    h = (
        jnp.dot(x_ref[...], w_ref[...], preferred_element_type=jnp.float32)
        + b_ref[...]
    )
    h_ref[...] = h
    os_ref[...] = jnp.dot(h, ws_ref[...], preferred_element_type=jnp.float32)
    od_ref[...] = jnp.dot(h, wd_ref[...], preferred_element_type=jnp.float32)


def _embed_proj(x, w, b, wsrc, wdst):
    return pl.pallas_call(
        _embed_proj_kernel,
        grid=(N // BM,),
        in_specs=[
            pl.BlockSpec((BM, ND), lambda i: (i, 0)),
            pl.BlockSpec((ND, H), lambda i: (0, 0)),
            pl.BlockSpec((1, H), lambda i: (0, 0)),
            pl.BlockSpec((H, 2 * H), lambda i: (0, 0)),
            pl.BlockSpec((H, 2 * H), lambda i: (0, 0)),
        ],
        out_specs=[
            pl.BlockSpec((BM, H), lambda i: (i, 0)),
            pl.BlockSpec((BM, 2 * H), lambda i: (i, 0)),
            pl.BlockSpec((BM, 2 * H), lambda i: (i, 0)),
        ],
        out_shape=[
            jax.ShapeDtypeStruct((N, H), jnp.float32),
            jax.ShapeDtypeStruct((N, 2 * H), jnp.float32),
            jax.ShapeDtypeStruct((N, 2 * H), jnp.float32),
        ],
        interpret=_INTERPRET,
    )(x, w, b.reshape(1, H), wsrc, wdst)


def _edge_nl_kernel(s_ref, ef_ref, we_ref, b_ref, o_ref):
    pre = (
        s_ref[...]
        + jnp.dot(ef_ref[...], we_ref[...], preferred_element_type=jnp.float32)
        + b_ref[...]
    )
    pc = pre[:, :H]
    pg = pre[:, H:]
    den = (1.0 + jnp.exp(-pc)) * (1.0 + jnp.exp(-pg))
    o_ref[...] = pc / den


def _edge_nl(s, ef, we, bcat):
    return pl.pallas_call(
        _edge_nl_kernel,
        grid=(E // BM,),
        in_specs=[
            pl.BlockSpec((BM, 2 * H), lambda i: (i, 0)),
            pl.BlockSpec((BM, ED), lambda i: (i, 0)),
            pl.BlockSpec((ED, 2 * H), lambda i: (0, 0)),
            pl.BlockSpec((1, 2 * H), lambda i: (0, 0)),
        ],
        out_specs=pl.BlockSpec((BM, H), lambda i: (i, 0)),
        out_shape=jax.ShapeDtypeStruct((E, H), jnp.float32),
        interpret=_INTERPRET,
    )(s, ef, we, bcat.reshape(1, 2 * H))


def _make_update_kernel(with_proj):
    def body(*refs):
        if with_proj:
            (hs, dp, g, b, hp, ws, wd,
             h_ref, os_ref, od_ref, x_scr, st_scr) = refs
        else:
            (hs, dp, g, b, hp, h_ref, x_scr, st_scr) = refs
        phase = pl.program_id(0)
        i = pl.program_id(1)

        @pl.when(phase == 0)
        def _():
            deg = jnp.clip(
                jnp.sum(dp[0] + dp[1], axis=-1, keepdims=True), 1.0, None
            )
            x = (hs[0] + hs[1]) / deg
            x_scr[pl.ds(i * BM, BM), :] = x

            @pl.when(i == 0)
            def _():
                st_scr[...] = jnp.zeros_like(st_scr)

            st_scr[0:1, :] += jnp.sum(x, axis=0, keepdims=True)
            st_scr[1:2, :] += jnp.sum(x * x, axis=0, keepdims=True)

        @pl.when(phase == 1)
        def _():
            mean = st_scr[0:1, :] / N
            var = st_scr[1:2, :] / N - mean * mean
            xn = (
                (x_scr[pl.ds(i * BM, BM), :] - mean)
                * lax.rsqrt(var + 1e-5) * g[...] + b[...]
            )
            t = hp[...] + xn
            h = t * (1.0 / (1.0 + jnp.exp(-t)))
            h_ref[...] = h
            if with_proj:
                os_ref[...] = jnp.dot(
                    h, ws[...], preferred_element_type=jnp.float32
                )
                od_ref[...] = jnp.dot(
                    h, wd[...], preferred_element_type=jnp.float32
                )

    in_specs = [
        pl.BlockSpec((NSC, BM, H), lambda p, i: (0, i, 0)),
        pl.BlockSpec((NSC, BM, H), lambda p, i: (0, i, 0)),
        pl.BlockSpec((1, H), lambda p, i: (0, 0)),
        pl.BlockSpec((1, H), lambda p, i: (0, 0)),
        pl.BlockSpec((BM, H), lambda p, i: (i, 0)),
    ]
    out_specs = [pl.BlockSpec((BM, H), lambda p, i: (i, 0))]
    out_shape = [jax.ShapeDtypeStruct((N, H), jnp.float32)]
    if with_proj:
        in_specs_x = in_specs + [
            pl.BlockSpec((H, 2 * H), lambda p, i: (0, 0)),
            pl.BlockSpec((H, 2 * H), lambda p, i: (0, 0)),
        ]
        out_specs_x = out_specs + [
            pl.BlockSpec((BM, 2 * H), lambda p, i: (i, 0)),
            pl.BlockSpec((BM, 2 * H), lambda p, i: (i, 0)),
        ]
        out_shape_x = out_shape + [
            jax.ShapeDtypeStruct((N, 2 * H), jnp.float32),
            jax.ShapeDtypeStruct((N, 2 * H), jnp.float32),
        ]
    else:
        in_specs_x, out_specs_x, out_shape_x = in_specs, out_specs, out_shape

    def run(hsum, dpart, bng, bnb, hprev, wsrc=None, wdst=None):
        args = [hsum, dpart, bng.reshape(1, H), bnb.reshape(1, H), hprev]
        if with_proj:
            args += [wsrc, wdst]
        return pl.pallas_call(
            body,
            grid=(2, N // BM),
            in_specs=in_specs_x,
            out_specs=out_specs_x,
            out_shape=out_shape_x,
            scratch_shapes=[
                pltpu.VMEM((N, H), jnp.float32),
                pltpu.VMEM((8, H), jnp.float32),
            ],
            interpret=_INTERPRET,
        )(*args)

    return run


def _make_gather_kernel():
    def body(src, dst, tsrc, tdst, out,
             idx_s0, idx_s1, idx_d0, idx_d1,
             rows_s0, rows_s1, rows_d0, rows_d1, sbuf0, sbuf1,
             sem_i0, sem_i1, sem_g0, sem_g1, sem_w0, sem_w1):
        cid = lax.axis_index("c")
        sid = lax.axis_index("s")
        wid = sid * NSC + cid
        base0 = wid * EPW

        idx_s = [idx_s0, idx_s1]
        idx_d = [idx_d0, idx_d1]
        rows_s = [rows_s0, rows_s1]
        rows_d = [rows_d0, rows_d1]
        sbuf = [sbuf0, sbuf1]
        sem_i = [sem_i0, sem_i1]
        sem_g = [sem_g0, sem_g1]
        sem_w = [sem_w0, sem_w1]

        def fire_idx(b, k):
            base = base0 + k * C
            pltpu.async_copy(src.at[pl.ds(base, C)], idx_s[b], sem_i[b])
            pltpu.async_copy(dst.at[pl.ds(base, C)], idx_d[b], sem_i[b])

        def wait_idx(b):
            pltpu.make_async_copy(src.at[pl.ds(0, C)], idx_s[b], sem_i[b]).wait()
            pltpu.make_async_copy(dst.at[pl.ds(0, C)], idx_d[b], sem_i[b]).wait()

        def fire_gather(b, k):
            pltpu.async_copy(tsrc.at[idx_s[b]], rows_s[b], sem_g[b])
            pltpu.async_copy(tdst.at[idx_d[b]], rows_d[b], sem_g[b])

        def wait_gather(b):
            pltpu.make_async_copy(tsrc.at[idx_s[b]], rows_s[b], sem_g[b]).wait()
            pltpu.make_async_copy(tdst.at[idx_d[b]], rows_d[b], sem_g[b]).wait()

        def fire_write(b, k):
            base = base0 + k * C
            pltpu.async_copy(sbuf[b], out.at[pl.ds(base, C)], sem_w[b])

        def wait_write(b):
            pltpu.make_async_copy(sbuf[b], out.at[pl.ds(0, C)], sem_w[b]).wait()

        def phase(b, k, ww, fi, fg):
            wait_gather(b)
            if ww:
                wait_write(b)
            if fi:
                fire_idx(b, k + 2)
            if fg:
                wait_idx(1 - b)
                fire_gather(1 - b, k + 1)
            rs, rd, sb = rows_s[b], rows_d[b], sbuf[b]

            def edge_body(e, _):
                for j in range(16):
                    cs = pl.ds(j * 16, 16)
                    sb[e, cs] = rs[e, cs] + rd[e, cs]
                return 0

            lax.fori_loop(0, C, edge_body, 0, unroll=2)
            fire_write(b, k)

        fire_idx(0, 0)
        wait_idx(0)
        fire_gather(0, 0)
        fire_idx(1, 1)
        phase(0, 0, False, True, True)
        phase(1, 1, False, True, True)

        def pair_body(i, _):
            k0 = 2 * i
            phase(0, k0, True, True, True)
            phase(1, k0 + 1, True, True, True)
            return 0

        lax.fori_loop(1, NCHUNK // 2 - 1, pair_body, 0)
        phase(0, NCHUNK - 2, True, False, True)
        phase(1, NCHUNK - 1, True, False, False)
        wait_write(0)
        wait_write(1)

    scratch = (
        [pltpu.VMEM((C,), jnp.int32) for _ in range(4)]
        + [pltpu.VMEM((C, 2 * H), jnp.float32) for _ in range(6)]
        + [pltpu.SemaphoreType.DMA for _ in range(6)]
    )
    mesh = plsc.VectorSubcoreMesh(
        core_axis_name="c", subcore_axis_name="s",
        num_cores=NSC, num_subcores=NSUB,
    )
    return pl.kernel(
        body,
        out_type=jax.ShapeDtypeStruct((E, 2 * H), jnp.float32),
        mesh=mesh,
        scratch_types=scratch,
        interpret=_INTERPRET,
    )


def _make_scatter_kernel():
    def body(dst, msg, out,
             idx_d0, idx_d1, mb0, mb1, acc, sem0, sem1):
        cid = lax.axis_index("c")
        sid = lax.axis_index("s")
        wid = sid * NSC + cid
        base0 = wid * EPW

        idx_d = [idx_d0, idx_d1]
        mb = [mb0, mb1]
        sem = [sem0, sem1]

        def zb(i, _):
            r = i // 8
            mb0[r, pl.ds((i % 8) * 16, 16)] = jnp.zeros((16,), jnp.float32)
            return 0

        lax.fori_loop(0, C * 8, zb, 0)
        for k in range(RPT // C):
            pltpu.sync_copy(mb0, acc.at[pl.ds(sid * RPT + k * C, C)])
        plsc.subcore_barrier()

        def fire_in(b, k):
            base = base0 + k * C
            pltpu.async_copy(dst.at[pl.ds(base, C)], idx_d[b], sem[b])
            pltpu.async_copy(msg.at[pl.ds(base, C)], mb[b], sem[b])

        def wait_in(b):
            pltpu.make_async_copy(dst.at[pl.ds(0, C)], idx_d[b], sem[b]).wait()
            pltpu.make_async_copy(msg.at[pl.ds(0, C)], mb[b], sem[b]).wait()

        def phase(b, k, ff):
            wait_in(b)
            if ff:
                fire_in(1 - b, k + 1)
            pltpu.sync_copy(mb[b], acc.at[idx_d[b]], add=True)

        fire_in(0, 0)

        def pair_body(i, _):
            k0 = 2 * i
            phase(0, k0, True)
            phase(1, k0 + 1, True)
            return 0

        lax.fori_loop(0, NCHUNK // 2 - 1, pair_body, 0)
        phase(0, NCHUNK - 2, True)
        phase(1, NCHUNK - 1, False)
        plsc.subcore_barrier()

        for k in range(RPT // C):
            r0 = sid * RPT + k * C
            pltpu.sync_copy(acc.at[pl.ds(r0, C)], mb0)
            pltpu.sync_copy(mb0, out.at[cid, pl.ds(r0, C)])

    scratch = (
        [pltpu.VMEM((C,), jnp.int32) for _ in range(2)]
        + [pltpu.VMEM((C, H), jnp.float32) for _ in range(2)]
        + [pltpu.VMEM_SHARED((NP_, H), jnp.float32)]
        + [pltpu.SemaphoreType.DMA for _ in range(2)]
    )
    mesh = plsc.VectorSubcoreMesh(
        core_axis_name="c", subcore_axis_name="s",
        num_cores=NSC, num_subcores=NSUB,
    )
    return pl.kernel(
        body,
        out_type=jax.ShapeDtypeStruct((NSC, NP_, H), jnp.float32),
        mesh=mesh,
        scratch_types=scratch,
        interpret=_INTERPRET,
    )


CD = 40


def _make_deg_kernel():
    def body(dst, outd, idx_d, ones_v, bnc, dacc, sem0):
        cid = lax.axis_index("c")
        sid = lax.axis_index("s")
        wid = sid * NSC + cid

        def zb(i, _):
            r = i // 8
            col = (i % 8) * 16
            bnc[r, pl.ds(col, 16)] = jnp.zeros((16,), jnp.float32)
            return 0

        lax.fori_loop(0, CD * 8, zb, 0)

        def ob(i, _):
            r = i // 8
            col = (i % 8) * 16
            ones_v[r, pl.ds(col, 16)] = jnp.full((16,), 1.0 / 128.0, jnp.float32)
            return 0

        lax.fori_loop(0, CD * 8, ob, 0)
        for k in range(RPT // CD):
            pltpu.sync_copy(bnc, dacc.at[pl.ds(sid * RPT + k * CD, CD)])
        plsc.subcore_barrier()

        def chunk_body(k, _):
            base = wid * EPW + k * CD
            pltpu.sync_copy(dst.at[pl.ds(base, CD)], idx_d)
            pltpu.sync_copy(ones_v, dacc.at[idx_d], add=True)
            return 0

        lax.fori_loop(0, EPW // CD, chunk_body, 0)
        plsc.subcore_barrier()
        for k in range(RPT // CD):
            r0 = sid * RPT + k * CD
            pltpu.sync_copy(dacc.at[pl.ds(r0, CD)], bnc)
            pltpu.sync_copy(bnc, outd.at[cid, pl.ds(r0, CD)])

    scratch = [
        pltpu.VMEM((CD,), jnp.int32),
        pltpu.VMEM((CD, H), jnp.float32),
        pltpu.VMEM((CD, H), jnp.float32),
        pltpu.VMEM_SHARED((NP_, H), jnp.float32),
        pltpu.SemaphoreType.DMA,
    ]
    mesh = plsc.VectorSubcoreMesh(
        core_axis_name="c", subcore_axis_name="s",
        num_cores=NSC, num_subcores=NSUB,
    )
    return pl.kernel(
        body,
        out_type=jax.ShapeDtypeStruct((NSC, NP_, H), jnp.float32),
        mesh=mesh,
        scratch_types=scratch,
        interpret=_INTERPRET,
    )


def _tail_kernel(h0, h1, h2, h3, w0, w1, w2, w3, jkb, oh,
                 fw0, fb0, bg, bb, fw1, fb1, o_ref, gsum, gmax, cnt):
    i = pl.program_id(0)
    hjk = (
        jnp.dot(h0[...], w0[...], preferred_element_type=jnp.float32)
        + jnp.dot(h1[...], w1[...], preferred_element_type=jnp.float32)
        + jnp.dot(h2[...], w2[...], preferred_element_type=jnp.float32)
        + jnp.dot(h3[...], w3[...], preferred_element_type=jnp.float32)
        + jkb[...]
    )

    @pl.when(i == 0)
    def _():
        gsum[...] = jnp.zeros_like(gsum)
        gmax[...] = jnp.full_like(gmax, -jnp.inf)
        cnt[...] = jnp.zeros_like(cnt)

    ohb = oh[...]
    gsum[...] += lax.dot_general(
        ohb, hjk, (((0,), (0,)), ((), ())),
        preferred_element_type=jnp.float32,
    )
    cnt[...] += lax.dot_general(
        ohb, jnp.ones_like(hjk), (((0,), (0,)), ((), ())),
        preferred_element_type=jnp.float32,
    )
    for b in range(B):
        m = ohb[:, b:b + 1] > 0.5
        mx = jnp.max(jnp.where(m, hjk, -jnp.inf), axis=0, keepdims=True)
        gmax[b:b + 1, :] = jnp.maximum(gmax[b:b + 1, :], mx)

    @pl.when(i == pl.num_programs(0) - 1)
    def _():
        c = jnp.clip(cnt[...], 1.0, None)
        gmean = gsum[...] / c
        gm = gmax[...]
        gm = jnp.where(gm > -1e30, gm, 0.0)
        g = jnp.concatenate([gmean, gm], axis=1)
        x = jnp.dot(g, fw0[...], preferred_element_type=jnp.float32) + fb0[...]
        mean = jnp.mean(x, axis=0, keepdims=True)
        var = jnp.mean(x * x, axis=0, keepdims=True) - mean * mean
        xn = (x - mean) * lax.rsqrt(var + 1e-5) * bg[...] + bb[...]
        xs = xn * (1.0 / (1.0 + jnp.exp(-xn)))
        o_ref[...] = (
            jnp.dot(xs, fw1[...], preferred_element_type=jnp.float32)
            + fb1[...]
        )


def _tail(states, jk_W, jk_b, onehot, fc0_W, fc0_b, bg, bb, fw1p, fb1p):
    w_specs = [pl.BlockSpec((H, H), lambda i: (0, 0)) for _ in range(4)]
    return pl.pallas_call(
        _tail_kernel,
        grid=(N // BM,),
        in_specs=(
            [pl.BlockSpec((BM, H), lambda i: (i, 0)) for _ in range(4)]
            + w_specs
            + [
                pl.BlockSpec((1, H), lambda i: (0, 0)),
                pl.BlockSpec((BM, B), lambda i: (i, 0)),
                pl.BlockSpec((2 * H, H), lambda i: (0, 0)),
                pl.BlockSpec((1, H), lambda i: (0, 0)),
                pl.BlockSpec((1, H), lambda i: (0, 0)),
                pl.BlockSpec((1, H), lambda i: (0, 0)),
                pl.BlockSpec((H, H), lambda i: (0, 0)),
                pl.BlockSpec((1, H), lambda i: (0, 0)),
            ]
        ),
        out_specs=pl.BlockSpec((B, H), lambda i: (0, 0)),
        out_shape=jax.ShapeDtypeStruct((B, H), jnp.float32),
        scratch_shapes=[
            pltpu.VMEM((B, H), jnp.float32),
            pltpu.VMEM((B, H), jnp.float32),
            pltpu.VMEM((B, H), jnp.float32),
        ],
        interpret=_INTERPRET,
    )(
        states[0], states[1], states[2], states[3],
        jk_W[0:H], jk_W[H:2 * H], jk_W[2 * H:3 * H], jk_W[3 * H:4 * H],
        jk_b.reshape(1, H), onehot,
        fc0_W, fc0_b.reshape(1, H), bg.reshape(1, H), bb.reshape(1, H),
        fw1p, fb1p.reshape(1, H),
    )


def kernel(node_feats, edge_index, edge_feats, batch, node_embed_W,
           node_embed_b, conv0_Wc, conv0_bc, conv0_Wg, conv0_bg, bn0_g,
           bn0_b, conv1_Wc, conv1_bc, conv1_Wg, conv1_bg, bn1_g, bn1_b,
           conv2_Wc, conv2_bc, conv2_Wg, conv2_bg, bn2_g, bn2_b, jk_W,
           jk_b, fc0_W, fc0_b, fc_bn_g, fc_bn_b, fc1_W, fc1_b):
    convs = [
        (conv0_Wc, conv0_bc, conv0_Wg, conv0_bg, bn0_g, bn0_b),
        (conv1_Wc, conv1_bc, conv1_Wg, conv1_bg, bn1_g, bn1_b),
        (conv2_Wc, conv2_bc, conv2_Wg, conv2_bg, bn2_g, bn2_b),
    ]
    src_idx = edge_index[0]
    dst_idx = edge_index[1]

    wsrc = [jnp.concatenate([Wc[0:H], Wg[0:H]], axis=1)
            for Wc, _, Wg, _, _, _ in convs]
    wdst = [jnp.concatenate([Wc[H:2 * H], Wg[H:2 * H]], axis=1)
            for Wc, _, Wg, _, _, _ in convs]
    we = [jnp.concatenate([Wc[2 * H:], Wg[2 * H:]], axis=1)
          for Wc, _, Wg, _, _, _ in convs]
    bcat = [jnp.concatenate([bc, bg]) for _, bc, _, bg, _, _ in convs]

    gat_k = _make_gather_kernel()
    sct_k = _make_scatter_kernel()
    upd_p = _make_update_kernel(True)
    upd_n = _make_update_kernel(False)

    h, tsrc, tdst = _embed_proj(
        node_feats, node_embed_W, node_embed_b, wsrc[0], wdst[0]
    )
    dpart = _make_deg_kernel()(dst_idx)
    states = [h]
    for i in range(NCONV):
        bng, bnb = convs[i][4], convs[i][5]
        s = gat_k(src_idx, dst_idx, tsrc, tdst)
        msg = _edge_nl(s, edge_feats, we[i], bcat[i])
        hsum = sct_k(dst_idx, msg)
        if i + 1 < NCONV:
            h, tsrc, tdst = upd_p(hsum, dpart, bng, bnb, h,
                                  wsrc[i + 1], wdst[i + 1])
        else:
            (h,) = upd_n(hsum, dpart, bng, bnb, h)
        states.append(h)

    onehot = (batch[:, None] == jnp.arange(B, dtype=jnp.int32)[None, :])
    onehot = onehot.astype(jnp.float32)
    fw1p = jnp.pad(fc1_W, ((0, 0), (0, H - 1)))
    fb1p = jnp.pad(fc1_b, (0, H - 1))
    out = _tail(states, jk_W, jk_b, onehot, fc0_W, fc0_b, fc_bn_g,
                fc_bn_b, fw1p, fb1p)
    return out[:, 0:1]


# speedup vs baseline: 2.3899x; 1.0783x over previous
import jax
import jax.numpy as jnp
from jax import lax
from jax.experimental import pallas as pl
from jax.experimental.pallas import tpu as pltpu
from jax.experimental.pallas import tpu_sc as plsc

N, E, ND, ED, H, NCONV, B = 10000, 320000, 128, 16, 128, 3, 16
NSC = 2
NSUB = 16
NW = NSC * NSUB
EPW = E // NW
C = 40
NCHUNK = EPW // C
NP_ = 10240
RPT = NP_ // NSUB
BM = 1000

_INTERPRET = False


def _embed_proj_kernel(x_ref, w_ref, b_ref, ws_ref, wd_ref,
                       h_ref, os_ref, od_ref):
    h = (
        jnp.dot(x_ref[...], w_ref[...], preferred_element_type=jnp.float32)
        + b_ref[...]
    )
    h_ref[...] = h
    os_ref[...] = jnp.dot(h, ws_ref[...], preferred_element_type=jnp.float32)
    od_ref[...] = jnp.dot(h, wd_ref[...], preferred_element_type=jnp.float32)


def _embed_proj(x, w, b, wsrc, wdst):
    return pl.pallas_call(
        _embed_proj_kernel,
        grid=(N // BM,),
        in_specs=[
            pl.BlockSpec((BM, ND), lambda i: (i, 0)),
            pl.BlockSpec((ND, H), lambda i: (0, 0)),
            pl.BlockSpec((1, H), lambda i: (0, 0)),
            pl.BlockSpec((H, 2 * H), lambda i: (0, 0)),
            pl.BlockSpec((H, 2 * H), lambda i: (0, 0)),
        ],
        out_specs=[
            pl.BlockSpec((BM, H), lambda i: (i, 0)),
            pl.BlockSpec((BM, 2 * H), lambda i: (i, 0)),
            pl.BlockSpec((BM, 2 * H), lambda i: (i, 0)),
        ],
        out_shape=[
            jax.ShapeDtypeStruct((N, H), jnp.float32),
            jax.ShapeDtypeStruct((N, 2 * H), jnp.float32),
            jax.ShapeDtypeStruct((N, 2 * H), jnp.float32),
        ],
        interpret=_INTERPRET,
    )(x, w, b.reshape(1, H), wsrc, wdst)


def _edge_nl_kernel(s_ref, ef_ref, we_ref, b_ref, o_ref):
    pre = (
        s_ref[...]
        + jnp.dot(ef_ref[...], we_ref[...], preferred_element_type=jnp.float32)
        + b_ref[...]
    )
    pc = pre[:, :H]
    pg = pre[:, H:]
    den = (1.0 + jnp.exp(-pc)) * (1.0 + jnp.exp(-pg))
    o_ref[...] = pc / den


def _edge_nl(s, ef, we, bcat):
    return pl.pallas_call(
        _edge_nl_kernel,
        grid=(E // BM,),
        in_specs=[
            pl.BlockSpec((BM, 2 * H), lambda i: (i, 0)),
            pl.BlockSpec((BM, ED), lambda i: (i, 0)),
            pl.BlockSpec((ED, 2 * H), lambda i: (0, 0)),
            pl.BlockSpec((1, 2 * H), lambda i: (0, 0)),
        ],
        out_specs=pl.BlockSpec((BM, H), lambda i: (i, 0)),
        out_shape=jax.ShapeDtypeStruct((E, H), jnp.float32),
        interpret=_INTERPRET,
    )(s, ef, we, bcat.reshape(1, 2 * H))


def _make_update_kernel(with_proj):
    def body(*refs):
        if with_proj:
            (hs, dp, g, b, hp, ws, wd,
             h_ref, os_ref, od_ref, x_scr, st_scr) = refs
        else:
            (hs, dp, g, b, hp, h_ref, x_scr, st_scr) = refs
        phase = pl.program_id(0)
        i = pl.program_id(1)

        @pl.when(phase == 0)
        def _():
            deg = jnp.clip(
                jnp.sum(dp[0] + dp[1], axis=-1, keepdims=True), 1.0, None
            )
            x = (hs[0] + hs[1]) / deg
            x_scr[pl.ds(i * BM, BM), :] = x

            @pl.when(i == 0)
            def _():
                st_scr[...] = jnp.zeros_like(st_scr)

            st_scr[0:1, :] += jnp.sum(x, axis=0, keepdims=True)
            st_scr[1:2, :] += jnp.sum(x * x, axis=0, keepdims=True)

        @pl.when(phase == 1)
        def _():
            mean = st_scr[0:1, :] / N
            var = st_scr[1:2, :] / N - mean * mean
            xn = (
                (x_scr[pl.ds(i * BM, BM), :] - mean)
                * lax.rsqrt(var + 1e-5) * g[...] + b[...]
            )
            t = hp[...] + xn
            h = t * (1.0 / (1.0 + jnp.exp(-t)))
            h_ref[...] = h
            if with_proj:
                os_ref[...] = jnp.dot(
                    h, ws[...], preferred_element_type=jnp.float32
                )
                od_ref[...] = jnp.dot(
                    h, wd[...], preferred_element_type=jnp.float32
                )

    in_specs = [
        pl.BlockSpec((NSC, BM, H), lambda p, i: (0, i, 0)),
        pl.BlockSpec((NSC, BM, H), lambda p, i: (0, i, 0)),
        pl.BlockSpec((1, H), lambda p, i: (0, 0)),
        pl.BlockSpec((1, H), lambda p, i: (0, 0)),
        pl.BlockSpec((BM, H), lambda p, i: (i, 0)),
    ]
    out_specs = [pl.BlockSpec((BM, H), lambda p, i: (i, 0))]
    out_shape = [jax.ShapeDtypeStruct((N, H), jnp.float32)]
    if with_proj:
        in_specs_x = in_specs + [
            pl.BlockSpec((H, 2 * H), lambda p, i: (0, 0)),
            pl.BlockSpec((H, 2 * H), lambda p, i: (0, 0)),
        ]
        out_specs_x = out_specs + [
            pl.BlockSpec((BM, 2 * H), lambda p, i: (i, 0)),
            pl.BlockSpec((BM, 2 * H), lambda p, i: (i, 0)),
        ]
        out_shape_x = out_shape + [
            jax.ShapeDtypeStruct((N, 2 * H), jnp.float32),
            jax.ShapeDtypeStruct((N, 2 * H), jnp.float32),
        ]
    else:
        in_specs_x, out_specs_x, out_shape_x = in_specs, out_specs, out_shape

    def run(hsum, dpart, bng, bnb, hprev, wsrc=None, wdst=None):
        args = [hsum, dpart, bng.reshape(1, H), bnb.reshape(1, H), hprev]
        if with_proj:
            args += [wsrc, wdst]
        return pl.pallas_call(
            body,
            grid=(2, N // BM),
            in_specs=in_specs_x,
            out_specs=out_specs_x,
            out_shape=out_shape_x,
            scratch_shapes=[
                pltpu.VMEM((N, H), jnp.float32),
                pltpu.VMEM((8, H), jnp.float32),
            ],
            interpret=_INTERPRET,
        )(*args)

    return run


CG = 80
NCHG = EPW // CG


def _make_gather_kernel():
    def body(src, dst, tsrc, tdst, out,
             idx_s0, idx_s1, idx_d0, idx_d1,
             rows_s0, rows_s1, rows_d0, rows_d1, sbuf0, sbuf1,
             sem_i0, sem_i1, sem_g0, sem_g1, sem_w0, sem_w1):
        cid = lax.axis_index("c")
        sid = lax.axis_index("s")
        wid = sid * NSC + cid
        base0 = wid * EPW

        idx_s = [idx_s0, idx_s1]
        idx_d = [idx_d0, idx_d1]
        rows_s = [rows_s0, rows_s1]
        rows_d = [rows_d0, rows_d1]
        sbuf = [sbuf0, sbuf1]
        sem_i = [sem_i0, sem_i1]
        sem_g = [sem_g0, sem_g1]
        sem_w = [sem_w0, sem_w1]

        def fire_idx(b, k):
            base = base0 + k * CG
            pltpu.async_copy(src.at[pl.ds(base, CG)], idx_s[b], sem_i[b])
            pltpu.async_copy(dst.at[pl.ds(base, CG)], idx_d[b], sem_i[b])

        def wait_idx(b):
            pltpu.make_async_copy(src.at[pl.ds(0, CG)], idx_s[b], sem_i[b]).wait()
            pltpu.make_async_copy(dst.at[pl.ds(0, CG)], idx_d[b], sem_i[b]).wait()

        def fire_gather(b, k):
            pltpu.async_copy(tsrc.at[idx_s[b]], rows_s[b], sem_g[b])
            pltpu.async_copy(tdst.at[idx_d[b]], rows_d[b], sem_g[b])

        def wait_gather(b):
            pltpu.make_async_copy(tsrc.at[idx_s[b]], rows_s[b], sem_g[b]).wait()
            pltpu.make_async_copy(tdst.at[idx_d[b]], rows_d[b], sem_g[b]).wait()

        def fire_write(b, k):
            base = base0 + k * CG
            pltpu.async_copy(sbuf[b], out.at[pl.ds(base, CG)], sem_w[b])

        def wait_write(b):
            pltpu.make_async_copy(sbuf[b], out.at[pl.ds(0, CG)], sem_w[b]).wait()

        def phase(b, k, ww, fi, fg):
            wait_gather(b)
            if ww:
                wait_write(b)
            if fi:
                fire_idx(b, k + 2)
            if fg:
                wait_idx(1 - b)
                fire_gather(1 - b, k + 1)
            rs, rd, sb = rows_s[b], rows_d[b], sbuf[b]

            def edge_body(e, _):
                for j in range(16):
                    cs = pl.ds(j * 16, 16)
                    sb[e, cs] = rs[e, cs] + rd[e, cs]
                return 0

            lax.fori_loop(0, CG, edge_body, 0, unroll=2)
            fire_write(b, k)

        fire_idx(0, 0)
        wait_idx(0)
        fire_gather(0, 0)
        fire_idx(1, 1)
        phase(0, 0, False, True, True)
        phase(1, 1, False, True, True)

        def pair_body(i, _):
            k0 = 2 * i
            phase(0, k0, True, True, True)
            phase(1, k0 + 1, True, True, True)
            return 0

        lax.fori_loop(1, (NCHG - 3) // 2, pair_body, 0)
        phase(0, NCHG - 3, True, True, True)
        phase(1, NCHG - 2, True, False, True)
        phase(0, NCHG - 1, True, False, False)
        wait_write(1)
        wait_write(0)

    scratch = (
        [pltpu.VMEM((CG,), jnp.int32) for _ in range(4)]
        + [pltpu.VMEM((CG, 2 * H), jnp.float32) for _ in range(6)]
        + [pltpu.SemaphoreType.DMA for _ in range(6)]
    )
    mesh = plsc.VectorSubcoreMesh(
        core_axis_name="c", subcore_axis_name="s",
        num_cores=NSC, num_subcores=NSUB,
    )
    return pl.kernel(
        body,
        out_type=jax.ShapeDtypeStruct((E, 2 * H), jnp.float32),
        mesh=mesh,
        scratch_types=scratch,
        interpret=_INTERPRET,
    )


def _make_scatter_kernel():
    def body(dst, msg, out,
             idx_d0, idx_d1, mb0, mb1, acc, sem0, sem1):
        cid = lax.axis_index("c")
        sid = lax.axis_index("s")
        wid = sid * NSC + cid
        base0 = wid * EPW

        idx_d = [idx_d0, idx_d1]
        mb = [mb0, mb1]
        sem = [sem0, sem1]

        def zb(i, _):
            r = i // 8
            mb0[r, pl.ds((i % 8) * 16, 16)] = jnp.zeros((16,), jnp.float32)
            return 0

        lax.fori_loop(0, C * 8, zb, 0)
        for k in range(RPT // C):
            pltpu.sync_copy(mb0, acc.at[pl.ds(sid * RPT + k * C, C)])
        plsc.subcore_barrier()

        def fire_in(b, k):
            base = base0 + k * C
            pltpu.async_copy(dst.at[pl.ds(base, C)], idx_d[b], sem[b])
            pltpu.async_copy(msg.at[pl.ds(base, C)], mb[b], sem[b])

        def wait_in(b):
            pltpu.make_async_copy(dst.at[pl.ds(0, C)], idx_d[b], sem[b]).wait()
            pltpu.make_async_copy(msg.at[pl.ds(0, C)], mb[b], sem[b]).wait()

        def phase(b, k, ff):
            wait_in(b)
            if ff:
                fire_in(1 - b, k + 1)
            pltpu.sync_copy(mb[b], acc.at[idx_d[b]], add=True)

        fire_in(0, 0)

        def pair_body(i, _):
            k0 = 2 * i
            phase(0, k0, True)
            phase(1, k0 + 1, True)
            return 0

        lax.fori_loop(0, NCHUNK // 2 - 1, pair_body, 0)
        phase(0, NCHUNK - 2, True)
        phase(1, NCHUNK - 1, False)
        plsc.subcore_barrier()

        for k in range(RPT // C):
            r0 = sid * RPT + k * C
            pltpu.sync_copy(acc.at[pl.ds(r0, C)], mb0)
            pltpu.sync_copy(mb0, out.at[cid, pl.ds(r0, C)])

    scratch = (
        [pltpu.VMEM((C,), jnp.int32) for _ in range(2)]
        + [pltpu.VMEM((C, H), jnp.float32) for _ in range(2)]
        + [pltpu.VMEM_SHARED((NP_, H), jnp.float32)]
        + [pltpu.SemaphoreType.DMA for _ in range(2)]
    )
    mesh = plsc.VectorSubcoreMesh(
        core_axis_name="c", subcore_axis_name="s",
        num_cores=NSC, num_subcores=NSUB,
    )
    return pl.kernel(
        body,
        out_type=jax.ShapeDtypeStruct((NSC, NP_, H), jnp.float32),
        mesh=mesh,
        scratch_types=scratch,
        interpret=_INTERPRET,
    )


CD = 40


def _make_deg_kernel():
    def body(dst, outd, idx_d, ones_v, bnc, dacc, sem0):
        cid = lax.axis_index("c")
        sid = lax.axis_index("s")
        wid = sid * NSC + cid

        def zb(i, _):
            r = i // 8
            col = (i % 8) * 16
            bnc[r, pl.ds(col, 16)] = jnp.zeros((16,), jnp.float32)
            return 0

        lax.fori_loop(0, CD * 8, zb, 0)

        def ob(i, _):
            r = i // 8
            col = (i % 8) * 16
            ones_v[r, pl.ds(col, 16)] = jnp.full((16,), 1.0 / 128.0, jnp.float32)
            return 0

        lax.fori_loop(0, CD * 8, ob, 0)
        for k in range(RPT // CD):
            pltpu.sync_copy(bnc, dacc.at[pl.ds(sid * RPT + k * CD, CD)])
        plsc.subcore_barrier()

        def chunk_body(k, _):
            base = wid * EPW + k * CD
            pltpu.sync_copy(dst.at[pl.ds(base, CD)], idx_d)
            pltpu.sync_copy(ones_v, dacc.at[idx_d], add=True)
            return 0

        lax.fori_loop(0, EPW // CD, chunk_body, 0)
        plsc.subcore_barrier()
        for k in range(RPT // CD):
            r0 = sid * RPT + k * CD
            pltpu.sync_copy(dacc.at[pl.ds(r0, CD)], bnc)
            pltpu.sync_copy(bnc, outd.at[cid, pl.ds(r0, CD)])

    scratch = [
        pltpu.VMEM((CD,), jnp.int32),
        pltpu.VMEM((CD, H), jnp.float32),
        pltpu.VMEM((CD, H), jnp.float32),
        pltpu.VMEM_SHARED((NP_, H), jnp.float32),
        pltpu.SemaphoreType.DMA,
    ]
    mesh = plsc.VectorSubcoreMesh(
        core_axis_name="c", subcore_axis_name="s",
        num_cores=NSC, num_subcores=NSUB,
    )
    return pl.kernel(
        body,
        out_type=jax.ShapeDtypeStruct((NSC, NP_, H), jnp.float32),
        mesh=mesh,
        scratch_types=scratch,
        interpret=_INTERPRET,
    )


def _tail_kernel(h0, h1, h2, h3, w0, w1, w2, w3, jkb, oh,
                 fw0, fb0, bg, bb, fw1, fb1, o_ref, gsum, gmax, cnt):
    i = pl.program_id(0)
    hjk = (
        jnp.dot(h0[...], w0[...], preferred_element_type=jnp.float32)
        + jnp.dot(h1[...], w1[...], preferred_element_type=jnp.float32)
        + jnp.dot(h2[...], w2[...], preferred_element_type=jnp.float32)
        + jnp.dot(h3[...], w3[...], preferred_element_type=jnp.float32)
        + jkb[...]
    )

    @pl.when(i == 0)
    def _():
        gsum[...] = jnp.zeros_like(gsum)
        gmax[...] = jnp.full_like(gmax, -jnp.inf)
        cnt[...] = jnp.zeros_like(cnt)

    ohb = oh[...]
    gsum[...] += lax.dot_general(
        ohb, hjk, (((0,), (0,)), ((), ())),
        preferred_element_type=jnp.float32,
    )
    cnt[...] += lax.dot_general(
        ohb, jnp.ones_like(hjk), (((0,), (0,)), ((), ())),
        preferred_element_type=jnp.float32,
    )
    for b in range(B):
        m = ohb[:, b:b + 1] > 0.5
        mx = jnp.max(jnp.where(m, hjk, -jnp.inf), axis=0, keepdims=True)
        gmax[b:b + 1, :] = jnp.maximum(gmax[b:b + 1, :], mx)

    @pl.when(i == pl.num_programs(0) - 1)
    def _():
        c = jnp.clip(cnt[...], 1.0, None)
        gmean = gsum[...] / c
        gm = gmax[...]
        gm = jnp.where(gm > -1e30, gm, 0.0)
        g = jnp.concatenate([gmean, gm], axis=1)
        x = jnp.dot(g, fw0[...], preferred_element_type=jnp.float32) + fb0[...]
        mean = jnp.mean(x, axis=0, keepdims=True)
        var = jnp.mean(x * x, axis=0, keepdims=True) - mean * mean
        xn = (x - mean) * lax.rsqrt(var + 1e-5) * bg[...] + bb[...]
        xs = xn * (1.0 / (1.0 + jnp.exp(-xn)))
        o_ref[...] = (
            jnp.dot(xs, fw1[...], preferred_element_type=jnp.float32)
            + fb1[...]
        )


def _tail(states, jk_W, jk_b, onehot, fc0_W, fc0_b, bg, bb, fw1p, fb1p):
    w_specs = [pl.BlockSpec((H, H), lambda i: (0, 0)) for _ in range(4)]
    return pl.pallas_call(
        _tail_kernel,
        grid=(N // BM,),
        in_specs=(
            [pl.BlockSpec((BM, H), lambda i: (i, 0)) for _ in range(4)]
            + w_specs
            + [
                pl.BlockSpec((1, H), lambda i: (0, 0)),
                pl.BlockSpec((BM, B), lambda i: (i, 0)),
                pl.BlockSpec((2 * H, H), lambda i: (0, 0)),
                pl.BlockSpec((1, H), lambda i: (0, 0)),
                pl.BlockSpec((1, H), lambda i: (0, 0)),
                pl.BlockSpec((1, H), lambda i: (0, 0)),
                pl.BlockSpec((H, H), lambda i: (0, 0)),
                pl.BlockSpec((1, H), lambda i: (0, 0)),
            ]
        ),
        out_specs=pl.BlockSpec((B, H), lambda i: (0, 0)),
        out_shape=jax.ShapeDtypeStruct((B, H), jnp.float32),
        scratch_shapes=[
            pltpu.VMEM((B, H), jnp.float32),
            pltpu.VMEM((B, H), jnp.float32),
            pltpu.VMEM((B, H), jnp.float32),
        ],
        interpret=_INTERPRET,
    )(
        states[0], states[1], states[2], states[3],
        jk_W[0:H], jk_W[H:2 * H], jk_W[2 * H:3 * H], jk_W[3 * H:4 * H],
        jk_b.reshape(1, H), onehot,
        fc0_W, fc0_b.reshape(1, H), bg.reshape(1, H), bb.reshape(1, H),
        fw1p, fb1p.reshape(1, H),
    )


def kernel(node_feats, edge_index, edge_feats, batch, node_embed_W,
           node_embed_b, conv0_Wc, conv0_bc, conv0_Wg, conv0_bg, bn0_g,
           bn0_b, conv1_Wc, conv1_bc, conv1_Wg, conv1_bg, bn1_g, bn1_b,
           conv2_Wc, conv2_bc, conv2_Wg, conv2_bg, bn2_g, bn2_b, jk_W,
           jk_b, fc0_W, fc0_b, fc_bn_g, fc_bn_b, fc1_W, fc1_b):
    convs = [
        (conv0_Wc, conv0_bc, conv0_Wg, conv0_bg, bn0_g, bn0_b),
        (conv1_Wc, conv1_bc, conv1_Wg, conv1_bg, bn1_g, bn1_b),
        (conv2_Wc, conv2_bc, conv2_Wg, conv2_bg, bn2_g, bn2_b),
    ]
    src_idx = edge_index[0]
    dst_idx = edge_index[1]

    wsrc = [jnp.concatenate([Wc[0:H], Wg[0:H]], axis=1)
            for Wc, _, Wg, _, _, _ in convs]
    wdst = [jnp.concatenate([Wc[H:2 * H], Wg[H:2 * H]], axis=1)
            for Wc, _, Wg, _, _, _ in convs]
    we = [jnp.concatenate([Wc[2 * H:], Wg[2 * H:]], axis=1)
          for Wc, _, Wg, _, _, _ in convs]
    bcat = [jnp.concatenate([bc, bg]) for _, bc, _, bg, _, _ in convs]

    gat_k = _make_gather_kernel()
    sct_k = _make_scatter_kernel()
    upd_p = _make_update_kernel(True)
    upd_n = _make_update_kernel(False)

    h, tsrc, tdst = _embed_proj(
        node_feats, node_embed_W, node_embed_b, wsrc[0], wdst[0]
    )
    dpart = _make_deg_kernel()(dst_idx)
    states = [h]
    for i in range(NCONV):
        bng, bnb = convs[i][4], convs[i][5]
        s = gat_k(src_idx, dst_idx, tsrc, tdst)
        msg = _edge_nl(s, edge_feats, we[i], bcat[i])
        hsum = sct_k(dst_idx, msg)
        if i + 1 < NCONV:
            h, tsrc, tdst = upd_p(hsum, dpart, bng, bnb, h,
                                  wsrc[i + 1], wdst[i + 1])
        else:
            (h,) = upd_n(hsum, dpart, bng, bnb, h)
        states.append(h)

    onehot = (batch[:, None] == jnp.arange(B, dtype=jnp.int32)[None, :])
    onehot = onehot.astype(jnp.float32)
    fw1p = jnp.pad(fc1_W, ((0, 0), (0, H - 1)))
    fb1p = jnp.pad(fc1_b, (0, H - 1))
    out = _tail(states, jk_W, jk_b, onehot, fc0_W, fc0_b, fc_bn_g,
                fc_bn_b, fw1p, fb1p)
    return out[:, 0:1]


# speedup vs baseline: 2.5792x; 1.0792x over previous
import jax
import jax.numpy as jnp
from jax import lax
from jax.experimental import pallas as pl
from jax.experimental.pallas import tpu as pltpu
from jax.experimental.pallas import tpu_sc as plsc

N, E, ND, ED, H, NCONV, B = 10000, 320000, 128, 16, 128, 3, 16
NSC = 2
NSUB = 16
NW = NSC * NSUB
EPW = E // NW
C = 40
NCHUNK = EPW // C
NP_ = 10240
RPT = NP_ // NSUB
BM = 1000

_INTERPRET = False


def _embed_proj_kernel(x_ref, w_ref, b_ref, ws_ref, wd_ref,
                       h_ref, os_ref, od_ref):
    h = (
        jnp.dot(x_ref[...], w_ref[...], preferred_element_type=jnp.float32)
        + b_ref[...]
    )
    h_ref[...] = h
    os_ref[...] = jnp.dot(h, ws_ref[...], preferred_element_type=jnp.float32)
    od_ref[...] = jnp.dot(h, wd_ref[...], preferred_element_type=jnp.float32)


def _embed_proj(x, w, b, wsrc, wdst):
    return pl.pallas_call(
        _embed_proj_kernel,
        grid=(N // BM,),
        in_specs=[
            pl.BlockSpec((BM, ND), lambda i: (i, 0)),
            pl.BlockSpec((ND, H), lambda i: (0, 0)),
            pl.BlockSpec((1, H), lambda i: (0, 0)),
            pl.BlockSpec((H, 2 * H), lambda i: (0, 0)),
            pl.BlockSpec((H, 2 * H), lambda i: (0, 0)),
        ],
        out_specs=[
            pl.BlockSpec((BM, H), lambda i: (i, 0)),
            pl.BlockSpec((BM, 2 * H), lambda i: (i, 0)),
            pl.BlockSpec((BM, 2 * H), lambda i: (i, 0)),
        ],
        out_shape=[
            jax.ShapeDtypeStruct((N, H), jnp.float32),
            jax.ShapeDtypeStruct((N, 2 * H), jnp.float32),
            jax.ShapeDtypeStruct((N, 2 * H), jnp.float32),
        ],
        interpret=_INTERPRET,
    )(x, w, b.reshape(1, H), wsrc, wdst)


def _edge_nl_kernel(s_ref, ef_ref, we_ref, b_ref, o_ref):
    pre = (
        s_ref[...]
        + jnp.dot(ef_ref[...], we_ref[...], preferred_element_type=jnp.float32)
        + b_ref[...]
    )
    pc = pre[:, :H]
    pg = pre[:, H:]
    den = (1.0 + jnp.exp(-pc)) * (1.0 + jnp.exp(-pg))
    o_ref[...] = pc / den


def _edge_nl(s, ef, we, bcat):
    return pl.pallas_call(
        _edge_nl_kernel,
        grid=(E // BM,),
        in_specs=[
            pl.BlockSpec((BM, 2 * H), lambda i: (i, 0)),
            pl.BlockSpec((BM, ED), lambda i: (i, 0)),
            pl.BlockSpec((ED, 2 * H), lambda i: (0, 0)),
            pl.BlockSpec((1, 2 * H), lambda i: (0, 0)),
        ],
        out_specs=pl.BlockSpec((BM, H), lambda i: (i, 0)),
        out_shape=jax.ShapeDtypeStruct((E, H), jnp.float32),
        interpret=_INTERPRET,
    )(s, ef, we, bcat.reshape(1, 2 * H))


def _make_update_kernel(with_proj):
    def body(*refs):
        if with_proj:
            (hs, dp, g, b, hp, ws, wd,
             h_ref, os_ref, od_ref, x_scr, st_scr) = refs
        else:
            (hs, dp, g, b, hp, h_ref, x_scr, st_scr) = refs
        phase = pl.program_id(0)
        i = pl.program_id(1)

        @pl.when(phase == 0)
        def _():
            deg = jnp.clip(
                jnp.sum(dp[0] + dp[1], axis=-1, keepdims=True), 1.0, None
            )
            x = (hs[0] + hs[1]) / deg
            x_scr[pl.ds(i * BM, BM), :] = x

            @pl.when(i == 0)
            def _():
                st_scr[...] = jnp.zeros_like(st_scr)

            st_scr[0:1, :] += jnp.sum(x, axis=0, keepdims=True)
            st_scr[1:2, :] += jnp.sum(x * x, axis=0, keepdims=True)

        @pl.when(phase == 1)
        def _():
            mean = st_scr[0:1, :] / N
            var = st_scr[1:2, :] / N - mean * mean
            xn = (
                (x_scr[pl.ds(i * BM, BM), :] - mean)
                * lax.rsqrt(var + 1e-5) * g[...] + b[...]
            )
            t = hp[...] + xn
            h = t * (1.0 / (1.0 + jnp.exp(-t)))
            h_ref[...] = h
            if with_proj:
                os_ref[...] = jnp.dot(
                    h, ws[...], preferred_element_type=jnp.float32
                )
                od_ref[...] = jnp.dot(
                    h, wd[...], preferred_element_type=jnp.float32
                )

    in_specs = [
        pl.BlockSpec((NSC, BM, H), lambda p, i: (0, i, 0)),
        pl.BlockSpec((NSC, BM, H), lambda p, i: (0, i, 0)),
        pl.BlockSpec((1, H), lambda p, i: (0, 0)),
        pl.BlockSpec((1, H), lambda p, i: (0, 0)),
        pl.BlockSpec((BM, H), lambda p, i: (i, 0)),
    ]
    out_specs = [pl.BlockSpec((BM, H), lambda p, i: (i, 0))]
    out_shape = [jax.ShapeDtypeStruct((N, H), jnp.float32)]
    if with_proj:
        in_specs_x = in_specs + [
            pl.BlockSpec((H, 2 * H), lambda p, i: (0, 0)),
            pl.BlockSpec((H, 2 * H), lambda p, i: (0, 0)),
        ]
        out_specs_x = out_specs + [
            pl.BlockSpec((BM, 2 * H), lambda p, i: (i, 0)),
            pl.BlockSpec((BM, 2 * H), lambda p, i: (i, 0)),
        ]
        out_shape_x = out_shape + [
            jax.ShapeDtypeStruct((N, 2 * H), jnp.float32),
            jax.ShapeDtypeStruct((N, 2 * H), jnp.float32),
        ]
    else:
        in_specs_x, out_specs_x, out_shape_x = in_specs, out_specs, out_shape

    def run(hsum, dpart, bng, bnb, hprev, wsrc=None, wdst=None):
        args = [hsum, dpart, bng.reshape(1, H), bnb.reshape(1, H), hprev]
        if with_proj:
            args += [wsrc, wdst]
        return pl.pallas_call(
            body,
            grid=(2, N // BM),
            in_specs=in_specs_x,
            out_specs=out_specs_x,
            out_shape=out_shape_x,
            scratch_shapes=[
                pltpu.VMEM((N, H), jnp.float32),
                pltpu.VMEM((8, H), jnp.float32),
            ],
            interpret=_INTERPRET,
        )(*args)

    return run


CG = 80
NCHG = EPW // CG


def _make_gather_kernel():
    def body(src, dst, tsrc, tdst, out,
             idx_s0, idx_s1, idx_d0, idx_d1,
             rows_s0, rows_s1, rows_d0, rows_d1, sbuf0, sbuf1,
             sem_i0, sem_i1, sem_g0, sem_g1, sem_w0, sem_w1):
        cid = lax.axis_index("c")
        sid = lax.axis_index("s")
        wid = sid * NSC + cid
        base0 = wid * EPW

        idx_s = [idx_s0, idx_s1]
        idx_d = [idx_d0, idx_d1]
        rows_s = [rows_s0, rows_s1]
        rows_d = [rows_d0, rows_d1]
        sbuf = [sbuf0, sbuf1]
        sem_i = [sem_i0, sem_i1]
        sem_g = [sem_g0, sem_g1]
        sem_w = [sem_w0, sem_w1]

        def fire_idx(b, k):
            base = base0 + k * CG
            pltpu.async_copy(src.at[pl.ds(base, CG)], idx_s[b], sem_i[b])
            pltpu.async_copy(dst.at[pl.ds(base, CG)], idx_d[b], sem_i[b])

        def wait_idx(b):
            pltpu.make_async_copy(src.at[pl.ds(0, CG)], idx_s[b], sem_i[b]).wait()
            pltpu.make_async_copy(dst.at[pl.ds(0, CG)], idx_d[b], sem_i[b]).wait()

        def fire_gather(b, k):
            pltpu.async_copy(tsrc.at[idx_s[b]], rows_s[b], sem_g[b])
            pltpu.async_copy(tdst.at[idx_d[b]], rows_d[b], sem_g[b])

        def wait_gather(b):
            pltpu.make_async_copy(tsrc.at[idx_s[b]], rows_s[b], sem_g[b]).wait()
            pltpu.make_async_copy(tdst.at[idx_d[b]], rows_d[b], sem_g[b]).wait()

        def fire_write(b, k):
            base = base0 + k * CG
            pltpu.async_copy(sbuf[b], out.at[pl.ds(base, CG)], sem_w[b])

        def wait_write(b):
            pltpu.make_async_copy(sbuf[b], out.at[pl.ds(0, CG)], sem_w[b]).wait()

        def phase(b, k, ww, fi, fg):
            wait_gather(b)
            if ww:
                wait_write(b)
            if fi:
                fire_idx(b, k + 2)
            if fg:
                wait_idx(1 - b)
                fire_gather(1 - b, k + 1)
            rs, rd, sb = rows_s[b], rows_d[b], sbuf[b]

            def edge_body(e, _):
                for j in range(16):
                    cs = pl.ds(j * 16, 16)
                    sb[e, cs] = rs[e, cs] + rd[e, cs]
                return 0

            lax.fori_loop(0, CG, edge_body, 0, unroll=2)
            fire_write(b, k)

        fire_idx(0, 0)
        wait_idx(0)
        fire_gather(0, 0)
        fire_idx(1, 1)
        phase(0, 0, False, True, True)
        phase(1, 1, False, True, True)

        def pair_body(i, _):
            k0 = 2 * i
            phase(0, k0, True, True, True)
            phase(1, k0 + 1, True, True, True)
            return 0

        lax.fori_loop(1, (NCHG - 3) // 2, pair_body, 0)
        phase(0, NCHG - 3, True, True, True)
        phase(1, NCHG - 2, True, False, True)
        phase(0, NCHG - 1, True, False, False)
        wait_write(1)
        wait_write(0)

    scratch = (
        [pltpu.VMEM((CG,), jnp.int32) for _ in range(4)]
        + [pltpu.VMEM((CG, 2 * H), jnp.float32) for _ in range(6)]
        + [pltpu.SemaphoreType.DMA for _ in range(6)]
    )
    mesh = plsc.VectorSubcoreMesh(
        core_axis_name="c", subcore_axis_name="s",
        num_cores=NSC, num_subcores=NSUB,
    )
    return pl.kernel(
        body,
        out_type=jax.ShapeDtypeStruct((E, 2 * H), jnp.float32),
        mesh=mesh,
        scratch_types=scratch,
        interpret=_INTERPRET,
    )


CS = 80
NCHS = EPW // CS


def _make_scatter_kernel():
    def body(dst, msg, out,
             idx_d0, idx_d1, mb0, mb1, acc, sem0, sem1):
        cid = lax.axis_index("c")
        sid = lax.axis_index("s")
        wid = sid * NSC + cid
        base0 = wid * EPW

        idx_d = [idx_d0, idx_d1]
        mb = [mb0, mb1]
        sem = [sem0, sem1]

        def zb(i, _):
            r = i // 8
            mb0[r, pl.ds((i % 8) * 16, 16)] = jnp.zeros((16,), jnp.float32)
            return 0

        lax.fori_loop(0, CS * 8, zb, 0)
        for k in range(RPT // CS):
            pltpu.sync_copy(mb0, acc.at[pl.ds(sid * RPT + k * CS, CS)])
        plsc.subcore_barrier()

        def fire_in(b, k):
            base = base0 + k * CS
            pltpu.async_copy(dst.at[pl.ds(base, CS)], idx_d[b], sem[b])
            pltpu.async_copy(msg.at[pl.ds(base, CS)], mb[b], sem[b])

        def wait_in(b):
            pltpu.make_async_copy(dst.at[pl.ds(0, CS)], idx_d[b], sem[b]).wait()
            pltpu.make_async_copy(msg.at[pl.ds(0, CS)], mb[b], sem[b]).wait()

        def phase(b, k, ff):
            wait_in(b)
            if ff:
                fire_in(1 - b, k + 1)
            pltpu.sync_copy(mb[b], acc.at[idx_d[b]], add=True)

        fire_in(0, 0)

        def pair_body(i, _):
            k0 = 2 * i
            phase(0, k0, True)
            phase(1, k0 + 1, True)
            return 0

        lax.fori_loop(0, (NCHS - 3) // 2, pair_body, 0)
        phase(0, NCHS - 3, True)
        phase(1, NCHS - 2, True)
        phase(0, NCHS - 1, False)
        plsc.subcore_barrier()

        for k in range(RPT // CS):
            r0 = sid * RPT + k * CS
            pltpu.sync_copy(acc.at[pl.ds(r0, CS)], mb0)
            pltpu.sync_copy(mb0, out.at[cid, pl.ds(r0, CS)])

    scratch = (
        [pltpu.VMEM((CS,), jnp.int32) for _ in range(2)]
        + [pltpu.VMEM((CS, H), jnp.float32) for _ in range(2)]
        + [pltpu.VMEM_SHARED((NP_, H), jnp.float32)]
        + [pltpu.SemaphoreType.DMA for _ in range(2)]
    )
    mesh = plsc.VectorSubcoreMesh(
        core_axis_name="c", subcore_axis_name="s",
        num_cores=NSC, num_subcores=NSUB,
    )
    return pl.kernel(
        body,
        out_type=jax.ShapeDtypeStruct((NSC, NP_, H), jnp.float32),
        mesh=mesh,
        scratch_types=scratch,
        interpret=_INTERPRET,
    )


CD = 80


def _make_deg_kernel():
    def body(dst, outd, idx_d, ones_v, bnc, dacc, sem0):
        cid = lax.axis_index("c")
        sid = lax.axis_index("s")
        wid = sid * NSC + cid

        def zb(i, _):
            r = i // 8
            col = (i % 8) * 16
            bnc[r, pl.ds(col, 16)] = jnp.zeros((16,), jnp.float32)
            return 0

        lax.fori_loop(0, CD * 8, zb, 0)

        def ob(i, _):
            r = i // 8
            col = (i % 8) * 16
            ones_v[r, pl.ds(col, 16)] = jnp.full((16,), 1.0 / 128.0, jnp.float32)
            return 0

        lax.fori_loop(0, CD * 8, ob, 0)
        for k in range(RPT // CD):
            pltpu.sync_copy(bnc, dacc.at[pl.ds(sid * RPT + k * CD, CD)])
        plsc.subcore_barrier()

        def chunk_body(k, _):
            base = wid * EPW + k * CD
            pltpu.sync_copy(dst.at[pl.ds(base, CD)], idx_d)
            pltpu.sync_copy(ones_v, dacc.at[idx_d], add=True)
            return 0

        lax.fori_loop(0, EPW // CD, chunk_body, 0)
        plsc.subcore_barrier()
        for k in range(RPT // CD):
            r0 = sid * RPT + k * CD
            pltpu.sync_copy(dacc.at[pl.ds(r0, CD)], bnc)
            pltpu.sync_copy(bnc, outd.at[cid, pl.ds(r0, CD)])

    scratch = [
        pltpu.VMEM((CD,), jnp.int32),
        pltpu.VMEM((CD, H), jnp.float32),
        pltpu.VMEM((CD, H), jnp.float32),
        pltpu.VMEM_SHARED((NP_, H), jnp.float32),
        pltpu.SemaphoreType.DMA,
    ]
    mesh = plsc.VectorSubcoreMesh(
        core_axis_name="c", subcore_axis_name="s",
        num_cores=NSC, num_subcores=NSUB,
    )
    return pl.kernel(
        body,
        out_type=jax.ShapeDtypeStruct((NSC, NP_, H), jnp.float32),
        mesh=mesh,
        scratch_types=scratch,
        interpret=_INTERPRET,
    )


def _tail_kernel(h0, h1, h2, h3, w0, w1, w2, w3, jkb, oh,
                 fw0, fb0, bg, bb, fw1, fb1, o_ref, gsum, gmax, cnt):
    i = pl.program_id(0)
    hjk = (
        jnp.dot(h0[...], w0[...], preferred_element_type=jnp.float32)
        + jnp.dot(h1[...], w1[...], preferred_element_type=jnp.float32)
        + jnp.dot(h2[...], w2[...], preferred_element_type=jnp.float32)
        + jnp.dot(h3[...], w3[...], preferred_element_type=jnp.float32)
        + jkb[...]
    )

    @pl.when(i == 0)
    def _():
        gsum[...] = jnp.zeros_like(gsum)
        gmax[...] = jnp.full_like(gmax, -jnp.inf)
        cnt[...] = jnp.zeros_like(cnt)

    ohb = oh[...]
    gsum[...] += lax.dot_general(
        ohb, hjk, (((0,), (0,)), ((), ())),
        preferred_element_type=jnp.float32,
    )
    cnt[...] += lax.dot_general(
        ohb, jnp.ones_like(hjk), (((0,), (0,)), ((), ())),
        preferred_element_type=jnp.float32,
    )
    for b in range(B):
        m = ohb[:, b:b + 1] > 0.5
        mx = jnp.max(jnp.where(m, hjk, -jnp.inf), axis=0, keepdims=True)
        gmax[b:b + 1, :] = jnp.maximum(gmax[b:b + 1, :], mx)

    @pl.when(i == pl.num_programs(0) - 1)
    def _():
        c = jnp.clip(cnt[...], 1.0, None)
        gmean = gsum[...] / c
        gm = gmax[...]
        gm = jnp.where(gm > -1e30, gm, 0.0)
        g = jnp.concatenate([gmean, gm], axis=1)
        x = jnp.dot(g, fw0[...], preferred_element_type=jnp.float32) + fb0[...]
        mean = jnp.mean(x, axis=0, keepdims=True)
        var = jnp.mean(x * x, axis=0, keepdims=True) - mean * mean
        xn = (x - mean) * lax.rsqrt(var + 1e-5) * bg[...] + bb[...]
        xs = xn * (1.0 / (1.0 + jnp.exp(-xn)))
        o_ref[...] = (
            jnp.dot(xs, fw1[...], preferred_element_type=jnp.float32)
            + fb1[...]
        )


def _tail(states, jk_W, jk_b, onehot, fc0_W, fc0_b, bg, bb, fw1p, fb1p):
    w_specs = [pl.BlockSpec((H, H), lambda i: (0, 0)) for _ in range(4)]
    return pl.pallas_call(
        _tail_kernel,
        grid=(N // BM,),
        in_specs=(
            [pl.BlockSpec((BM, H), lambda i: (i, 0)) for _ in range(4)]
            + w_specs
            + [
                pl.BlockSpec((1, H), lambda i: (0, 0)),
                pl.BlockSpec((BM, B), lambda i: (i, 0)),
                pl.BlockSpec((2 * H, H), lambda i: (0, 0)),
                pl.BlockSpec((1, H), lambda i: (0, 0)),
                pl.BlockSpec((1, H), lambda i: (0, 0)),
                pl.BlockSpec((1, H), lambda i: (0, 0)),
                pl.BlockSpec((H, H), lambda i: (0, 0)),
                pl.BlockSpec((1, H), lambda i: (0, 0)),
            ]
        ),
        out_specs=pl.BlockSpec((B, H), lambda i: (0, 0)),
        out_shape=jax.ShapeDtypeStruct((B, H), jnp.float32),
        scratch_shapes=[
            pltpu.VMEM((B, H), jnp.float32),
            pltpu.VMEM((B, H), jnp.float32),
            pltpu.VMEM((B, H), jnp.float32),
        ],
        interpret=_INTERPRET,
    )(
        states[0], states[1], states[2], states[3],
        jk_W[0:H], jk_W[H:2 * H], jk_W[2 * H:3 * H], jk_W[3 * H:4 * H],
        jk_b.reshape(1, H), onehot,
        fc0_W, fc0_b.reshape(1, H), bg.reshape(1, H), bb.reshape(1, H),
        fw1p, fb1p.reshape(1, H),
    )


def kernel(node_feats, edge_index, edge_feats, batch, node_embed_W,
           node_embed_b, conv0_Wc, conv0_bc, conv0_Wg, conv0_bg, bn0_g,
           bn0_b, conv1_Wc, conv1_bc, conv1_Wg, conv1_bg, bn1_g, bn1_b,
           conv2_Wc, conv2_bc, conv2_Wg, conv2_bg, bn2_g, bn2_b, jk_W,
           jk_b, fc0_W, fc0_b, fc_bn_g, fc_bn_b, fc1_W, fc1_b):
    convs = [
        (conv0_Wc, conv0_bc, conv0_Wg, conv0_bg, bn0_g, bn0_b),
        (conv1_Wc, conv1_bc, conv1_Wg, conv1_bg, bn1_g, bn1_b),
        (conv2_Wc, conv2_bc, conv2_Wg, conv2_bg, bn2_g, bn2_b),
    ]
    src_idx = edge_index[0]
    dst_idx = edge_index[1]

    wsrc = [jnp.concatenate([Wc[0:H], Wg[0:H]], axis=1)
            for Wc, _, Wg, _, _, _ in convs]
    wdst = [jnp.concatenate([Wc[H:2 * H], Wg[H:2 * H]], axis=1)
            for Wc, _, Wg, _, _, _ in convs]
    we = [jnp.concatenate([Wc[2 * H:], Wg[2 * H:]], axis=1)
          for Wc, _, Wg, _, _, _ in convs]
    bcat = [jnp.concatenate([bc, bg]) for _, bc, _, bg, _, _ in convs]

    gat_k = _make_gather_kernel()
    sct_k = _make_scatter_kernel()
    upd_p = _make_update_kernel(True)
    upd_n = _make_update_kernel(False)

    h, tsrc, tdst = _embed_proj(
        node_feats, node_embed_W, node_embed_b, wsrc[0], wdst[0]
    )
    dpart = _make_deg_kernel()(dst_idx)
    states = [h]
    for i in range(NCONV):
        bng, bnb = convs[i][4], convs[i][5]
        s = gat_k(src_idx, dst_idx, tsrc, tdst)
        msg = _edge_nl(s, edge_feats, we[i], bcat[i])
        hsum = sct_k(dst_idx, msg)
        if i + 1 < NCONV:
            h, tsrc, tdst = upd_p(hsum, dpart, bng, bnb, h,
                                  wsrc[i + 1], wdst[i + 1])
        else:
            (h,) = upd_n(hsum, dpart, bng, bnb, h)
        states.append(h)

    onehot = (batch[:, None] == jnp.arange(B, dtype=jnp.int32)[None, :])
    onehot = onehot.astype(jnp.float32)
    fw1p = jnp.pad(fc1_W, ((0, 0), (0, H - 1)))
    fb1p = jnp.pad(fc1_b, (0, H - 1))
    out = _tail(states, jk_W, jk_b, onehot, fc0_W, fc0_b, fc_bn_g,
                fc_bn_b, fw1p, fb1p)
    return out[:, 0:1]
